# Initial kernel scaffold; baseline (speedup 1.0000x reference)
import jax
import jax.numpy as jnp
from jax.experimental import pallas as pl


def kernel(x, edge_index, edge_attr, batch, W1, a_src1, a_dst1, We1, a_edge1, b1, g1, be1, W2, a_src2, a_dst2, We2, a_edge2, b2, g2, be2, Wf, bf):
    raise NotImplementedError("write your pallas kernel here")


# speedup vs baseline: 17.2121x; 17.2121x over previous
import functools

import jax
import jax.numpy as jnp
from jax import lax
from jax.experimental import pallas as pl
from jax.experimental.pallas import tpu as pltpu
from jax.experimental.pallas import tpu_sc as plsc

N = 10000
E = 320000
D_IN = 128
HID = 32
HEADS = 8
G = 64

NP = 10240
NB = 10
BLK = N // NB
BLKP = NP // NB


def _k1_body(x_ref, w00_ref, v00_ref, w01_ref, v01_ref,
             w10_ref, v10_ref, w11_ref, v11_ref, vd_ref, c16_ref,
             t00_ref, t01_ref, t10_ref, t11_ref, dm_ref):
    xb = x_ref[...]
    f32 = jnp.float32
    for w_ref, v_ref, t_ref in ((w00_ref, v00_ref, t00_ref),
                                (w01_ref, v01_ref, t01_ref),
                                (w10_ref, v10_ref, t10_ref),
                                (w11_ref, v11_ref, t11_ref)):
        t_ref[:, 0:64] = jnp.dot(xb, w_ref[...], preferred_element_type=f32)
        t_ref[:, 64:80] = jnp.dot(xb, v_ref[...], preferred_element_type=f32)
    dm_ref[...] = (jnp.dot(xb, vd_ref[...], preferred_element_type=f32)
                   + c16_ref[0:1, :])


def _k1(x, wpieces, vpieces, vd, c16):
    f32 = jnp.float32
    wspec = pl.BlockSpec((128, 64), lambda i: (0, 0))
    vspec = pl.BlockSpec((128, 16), lambda i: (0, 0))
    tspec = pl.BlockSpec((BLK, 80), lambda i: (i, 0))
    tshape = jax.ShapeDtypeStruct((N, 80), f32)
    return pl.pallas_call(
        _k1_body,
        grid=(NB,),
        in_specs=[
            pl.BlockSpec((BLK, 128), lambda i: (i, 0)),
            wspec, vspec, wspec, vspec, wspec, vspec, wspec, vspec,
            pl.BlockSpec((128, 16), lambda i: (0, 0)),
            pl.BlockSpec((8, 16), lambda i: (0, 0)),
        ],
        out_specs=[tspec, tspec, tspec, tspec,
                   pl.BlockSpec((BLK, 16), lambda i: (i, 0))],
        out_shape=[tshape, tshape, tshape, tshape,
                   jax.ShapeDtypeStruct((N, 16), f32)],
    )(x, wpieces[0], vpieces[0], wpieces[1], vpieces[1],
      wpieces[2], vpieces[2], wpieces[3], vpieces[3], vd, c16)


def _bn_stats_body(u0, u1, u2, u3, d0, d1, d2, d3, b0, b1r, b2r_, b3,
                   s1_ref, s2_ref):
    i = pl.program_id(0)

    @pl.when(i == 0)
    def _():
        s1_ref[...] = jnp.zeros_like(s1_ref)
        s2_ref[...] = jnp.zeros_like(s2_ref)

    for p, (un_ref, de_ref, b_ref) in enumerate(
            ((u0, d0, b0), (u1, d1, b1r), (u2, d2, b2r_), (u3, d3, b3))):
        for k in range(2):
            sl = slice(k * 32, (k + 1) * 32)
            v = (un_ref[:, sl] / (de_ref[:, k:k + 1] + 1e-16)
                 + b_ref[0:1, sl])
            osl = slice(p * 64 + k * 32, p * 64 + (k + 1) * 32)
            s1_ref[0:1, osl] += jnp.sum(v, axis=0, keepdims=True)
            s2_ref[0:1, osl] += jnp.sum(v * v, axis=0, keepdims=True)


def _k3a(uns, des, bs):
    f32 = jnp.float32
    uspec = pl.BlockSpec((BLKP, 64), lambda i: (i, 0))
    dspec = pl.BlockSpec((BLKP, 16), lambda i: (i, 0))
    bspec = pl.BlockSpec((1, 64), lambda i: (0, 0))
    return pl.pallas_call(
        _bn_stats_body,
        grid=(NB,),
        in_specs=[uspec] * 4 + [dspec] * 4 + [bspec] * 4,
        out_specs=[
            pl.BlockSpec((8, 256), lambda i: (0, 0)),
            pl.BlockSpec((8, 256), lambda i: (0, 0)),
        ],
        out_shape=[
            jax.ShapeDtypeStruct((8, 256), f32),
            jax.ShapeDtypeStruct((8, 256), f32),
        ],
    )(*uns, *des, *bs)


def _k3b_body(u0, u1, u2, u3, d0, d1, d2, d3, b0, b1r, b2r_, b3,
              sc0, sc1, sc2, sc3, sh0, sh1, sh2, sh3, w0, w1, w2_, w3,
              a2_ref, ad2_ref, c2r_ref, t2_ref, d2_ref):
    f32 = jnp.float32
    h2 = jnp.zeros((BLKP, HID), f32)
    for un_ref, de_ref, b_ref, sc_ref, sh_ref, w2_ref in (
            (u0, d0, b0, sc0, sh0, w0), (u1, d1, b1r, sc1, sh1, w1),
            (u2, d2, b2r_, sc2, sh2, w2_), (u3, d3, b3, sc3, sh3, w3)):
        for k in range(2):
            sl = slice(k * 32, (k + 1) * 32)
            v = (un_ref[:, sl] / (de_ref[:, k:k + 1] + 1e-16)
                 + b_ref[0:1, sl])
            v = v * sc_ref[0:1, sl] + sh_ref[0:1, sl]
            v = jnp.maximum(v, 0.0)
            h2 = h2 + jnp.dot(v, w2_ref[sl, :], preferred_element_type=f32)
    t2_ref[:, 0:32] = h2
    t2_ref[:, 32:48] = jnp.dot(h2, a2_ref[...], preferred_element_type=f32)
    d2_ref[...] = (jnp.dot(h2, ad2_ref[...], preferred_element_type=f32)
                   + c2r_ref[0:1, :])


def _k3b(uns, des, bs, scs, shs, w2s, a2, ad2, c2r):
    f32 = jnp.float32
    uspec = pl.BlockSpec((BLKP, 64), lambda i: (i, 0))
    dspec = pl.BlockSpec((BLKP, 16), lambda i: (i, 0))
    bspec = pl.BlockSpec((1, 64), lambda i: (0, 0))
    wspec = pl.BlockSpec((64, 32), lambda i: (0, 0))
    return pl.pallas_call(
        _k3b_body,
        grid=(NB,),
        in_specs=[uspec] * 4 + [dspec] * 4 + [bspec] * 12 + [wspec] * 4 + [
            pl.BlockSpec((32, 16), lambda i: (0, 0)),
            pl.BlockSpec((32, 16), lambda i: (0, 0)),
            pl.BlockSpec((8, 16), lambda i: (0, 0)),
        ],
        out_specs=[
            pl.BlockSpec((BLKP, 48), lambda i: (i, 0)),
            pl.BlockSpec((BLKP, 16), lambda i: (i, 0)),
        ],
        out_shape=[
            jax.ShapeDtypeStruct((NP, 48), f32),
            jax.ShapeDtypeStruct((NP, 16), f32),
        ],
    )(*uns, *des, *bs, *scs, *shs, *w2s, a2, ad2, c2r)


def _bn2_stats_body(una_ref, unb_ref, dea_ref, deb_ref, b2_ref,
                    s1_ref, s2_ref):
    i = pl.program_id(0)

    @pl.when(i == 0)
    def _():
        s1_ref[...] = jnp.zeros_like(s1_ref)
        s2_ref[...] = jnp.zeros_like(s2_ref)

    de = dea_ref[:, 0:1] + deb_ref[:, 0:1] + 1e-16
    v = (una_ref[...] + unb_ref[...]) / de + b2_ref[0:1, :]
    s1_ref[0:1, :] += jnp.sum(v, axis=0, keepdims=True)
    s2_ref[0:1, :] += jnp.sum(v * v, axis=0, keepdims=True)


def _k5a(un2_a, un2_b, de2_a, de2_b, b2r):
    f32 = jnp.float32
    return pl.pallas_call(
        _bn2_stats_body,
        grid=(NB,),
        in_specs=[
            pl.BlockSpec((BLKP, 32), lambda i: (i, 0)),
            pl.BlockSpec((BLKP, 32), lambda i: (i, 0)),
            pl.BlockSpec((BLKP, 16), lambda i: (i, 0)),
            pl.BlockSpec((BLKP, 16), lambda i: (i, 0)),
            pl.BlockSpec((1, 32), lambda i: (0, 0)),
        ],
        out_specs=[
            pl.BlockSpec((8, 32), lambda i: (0, 0)),
            pl.BlockSpec((8, 32), lambda i: (0, 0)),
        ],
        out_shape=[
            jax.ShapeDtypeStruct((8, 32), f32),
            jax.ShapeDtypeStruct((8, 32), f32),
        ],
    )(un2_a, un2_b, de2_a, de2_b, b2r)


def _pool_body(una_ref, unb_ref, dea_ref, deb_ref, b2_ref, sc2_ref, sh2_ref,
               batch_ref, wf_ref, bf_ref, p_ref, cnt_ref, out_ref):
    i = pl.program_id(0)
    f32 = jnp.float32

    @pl.when(i == 0)
    def _():
        p_ref[...] = jnp.zeros_like(p_ref)
        cnt_ref[...] = jnp.zeros_like(cnt_ref)

    de = dea_ref[:, 0:1] + deb_ref[:, 0:1] + 1e-16
    v = (una_ref[...] + unb_ref[...]) / de + b2_ref[0:1, :]
    h = jnp.maximum(v * sc2_ref[0:1, :] + sh2_ref[0:1, :], 0.0)
    oh = (batch_ref[...] == lax.broadcasted_iota(jnp.int32, (BLKP, G), 1)
          ).astype(f32)
    dn = (((0,), (0,)), ((), ()))
    p_ref[...] += lax.dot_general(oh, h, dn, preferred_element_type=f32)
    cnt_ref[...] += lax.dot_general(oh, jnp.ones((BLKP, 8), f32), dn,
                                    preferred_element_type=f32)

    @pl.when(i == NB - 1)
    def _():
        pooled = p_ref[...] / jnp.maximum(cnt_ref[:, 0:1], 1.0)
        out_ref[...] = (jnp.dot(pooled, wf_ref[...],
                                preferred_element_type=f32) + bf_ref[0:1, :])


def _k5b(un2_a, un2_b, de2_a, de2_b, b2r, sc2, sh2, batch2d, wf, bfr):
    f32 = jnp.float32
    row32 = pl.BlockSpec((1, 32), lambda i: (0, 0))
    return pl.pallas_call(
        _pool_body,
        grid=(NB,),
        in_specs=[
            pl.BlockSpec((BLKP, 32), lambda i: (i, 0)),
            pl.BlockSpec((BLKP, 32), lambda i: (i, 0)),
            pl.BlockSpec((BLKP, 16), lambda i: (i, 0)),
            pl.BlockSpec((BLKP, 16), lambda i: (i, 0)),
            row32, row32, row32,
            pl.BlockSpec((BLKP, 1), lambda i: (i, 0)),
            pl.BlockSpec((32, 32), lambda i: (0, 0)),
            row32,
        ],
        out_specs=[
            pl.BlockSpec((G, 32), lambda i: (0, 0)),
            pl.BlockSpec((G, 8), lambda i: (0, 0)),
            pl.BlockSpec((G, 32), lambda i: (0, 0)),
        ],
        out_shape=[
            jax.ShapeDtypeStruct((G, 32), f32),
            jax.ShapeDtypeStruct((G, 8), f32),
            jax.ShapeDtypeStruct((G, 32), f32),
        ],
    )(un2_a, un2_b, de2_a, de2_b, b2r, sc2, sh2, batch2d, wf, bfr)


_MESH = plsc.VectorSubcoreMesh(core_axis_name="c", subcore_axis_name="s")


def _vtake(x, idx):
    return jnp.take_along_axis(x, idx, axis=0, mode="promise_in_bounds")


def _splat16(i):
    return jnp.full((16,), i, jnp.int32)

EPT1 = E // 16
C1 = 160
ROWS_PT = NP // 16
EPW2 = E // 32
C2 = 400


def _sc1_body(src, dst, ea, t00, t01, t10, t11, dm,
              un0, un1, un2, un3, de0, de1, de2, de3,
              zb, zb16, srcv, dstv, eav, trows, dmrows, wh, w_rm,
              un_acc, de_acc):
    f32 = jnp.float32
    z16 = jnp.zeros((16,), f32)
    cid = lax.axis_index("c")
    sid = lax.axis_index("s")
    iota16 = lax.iota(jnp.int32, 16)
    lane_lt2 = iota16 < 2

    def zrow(i, carry):
        for j in range(4):
            zb[i, pl.ds(j * 16, 16)] = z16
        zb16[i] = z16
        return carry

    lax.fori_loop(0, 128, zrow, 0)

    r0 = sid * ROWS_PT
    ebase = sid * EPT1

    for q, (tac, tbc, un_c0, un_c1, de_c0, de_c1) in enumerate(
            ((t00, t10, un0, un2, de0, de2),
             (t01, t11, un1, un3, de1, de3))):
        hco = cid * 4 + q * 2
        idx_ad = (iota16 + hco) & 15
        idx_c = (iota16 + (hco + 8)) & 15

        for jj in range(5):
            pltpu.sync_copy(zb, un_acc.at[pl.ds(r0 + jj * 128, 128)])
            pltpu.sync_copy(zb16, de_acc.at[pl.ds(r0 + jj * 128, 128)])
        plsc.subcore_barrier()

        def chunk(j, carry):
            base = ebase + j * C1
            pltpu.sync_copy(src.at[pl.ds(base, C1)], srcv)
            pltpu.sync_copy(dst.at[pl.ds(base, C1)], dstv)
            pltpu.sync_copy(ea.at[pl.ds(base, C1)], eav)

            @pl.when(cid == 0)
            def _():
                pltpu.sync_copy(tac.at[srcv], trows)

            @pl.when(cid == 1)
            def _():
                pltpu.sync_copy(tbc.at[srcv], trows)

            pltpu.sync_copy(dm.at[dstv], dmrows)

            def grp(g, carry2):
                ev = eav[pl.ds(g * 16, 16)]
                for i in range(16):
                    e = g * 16 + i
                    av = trows[e, pl.ds(64, 16)]
                    dv = dmrows[e, pl.ds(0, 16)]
                    ea_e = _vtake(ev, _splat16(i))
                    ad = _vtake(dv, idx_ad)
                    cv = _vtake(dv, idx_c)
                    a = av + ad + ea_e * cv
                    a = jnp.where(a >= 0.0, a, 0.2 * a)
                    w = jnp.exp(a)
                    w2l = jnp.where(lane_lt2, w, 0.0)
                    w_rm[e] = w2l
                    wb = [_vtake(w2l, _splat16(h)) for h in range(2)]
                    for r in range(4):
                        sl = pl.ds(r * 16, 16)
                        wh[e, sl] = trows[e, sl] * wb[r // 2]
                return carry2

            lax.fori_loop(0, C1 // 16, grp, 0)

            pltpu.sync_copy(wh, un_acc.at[dstv], add=True)
            pltpu.sync_copy(w_rm, de_acc.at[dstv], add=True)
            return carry

        lax.fori_loop(0, EPT1 // C1, chunk, 0)
        plsc.subcore_barrier()

        for jj in range(5):
            rr = r0 + jj * 128

            @pl.when(cid == 0)
            def _():
                pltpu.sync_copy(un_acc.at[pl.ds(rr, 128)],
                                un_c0.at[pl.ds(rr, 128)])
                pltpu.sync_copy(de_acc.at[pl.ds(rr, 128)],
                                de_c0.at[pl.ds(rr, 128)])

            @pl.when(cid == 1)
            def _():
                pltpu.sync_copy(un_acc.at[pl.ds(rr, 128)],
                                un_c1.at[pl.ds(rr, 128)])
                pltpu.sync_copy(de_acc.at[pl.ds(rr, 128)],
                                de_c1.at[pl.ds(rr, 128)])


def _sc1(src, dst, ea, t00, t01, t10, t11, dm):
    f32 = jnp.float32
    k = pl.kernel(
        _sc1_body,
        mesh=_MESH,
        compiler_params=pltpu.CompilerParams(use_tc_tiling_on_sc=False),
        out_type=[jax.ShapeDtypeStruct((NP, 64), f32)] * 4
        + [jax.ShapeDtypeStruct((NP, 16), f32)] * 4,
        scratch_types=[
            pltpu.VMEM((128, 64), f32),
            pltpu.VMEM((128, 16), f32),
            pltpu.VMEM((C1,), jnp.int32),
            pltpu.VMEM((C1,), jnp.int32),
            pltpu.VMEM((C1,), f32),
            pltpu.VMEM((C1, 80), f32),
            pltpu.VMEM((C1, 16), f32),
            pltpu.VMEM((C1, 64), f32),
            pltpu.VMEM((C1, 16), f32),
            pltpu.VMEM_SHARED((NP, 64), f32),
            pltpu.VMEM_SHARED((NP, 16), f32),
        ],
    )
    return k(src, dst, ea, t00, t01, t10, t11, dm)


def _sc2_body(src, dst, ea, t2, d2,
              un2_a, un2_b, de2_a, de2_b,
              zb32, zb16, srcv, dstv, eav, t2rows, d2rows, wh2, w_rm,
              un_acc, de_acc):
    f32 = jnp.float32
    z16 = jnp.zeros((16,), f32)
    cid = lax.axis_index("c")
    sid = lax.axis_index("s")
    wid = cid * 16 + sid
    iota16 = lax.iota(jnp.int32, 16)
    lane_is0 = iota16 == 0

    def zrow(i, carry):
        zb32[i, pl.ds(0, 16)] = z16
        zb32[i, pl.ds(16, 16)] = z16
        zb16[i] = z16
        return carry

    lax.fori_loop(0, 128, zrow, 0)

    r0 = sid * ROWS_PT
    for jj in range(5):
        pltpu.sync_copy(zb32, un_acc.at[pl.ds(r0 + jj * 128, 128)])
        pltpu.sync_copy(zb16, de_acc.at[pl.ds(r0 + jj * 128, 128)])
    plsc.subcore_barrier()

    ebase = wid * EPW2

    def chunk(j, carry):
        base = ebase + j * C2
        pltpu.sync_copy(src.at[pl.ds(base, C2)], srcv)
        pltpu.sync_copy(dst.at[pl.ds(base, C2)], dstv)
        pltpu.sync_copy(ea.at[pl.ds(base, C2)], eav)
        pltpu.sync_copy(t2.at[srcv], t2rows)
        pltpu.sync_copy(d2.at[dstv], d2rows)

        def grp(g, carry2):
            ev = eav[pl.ds(g * 16, 16)]
            for i in range(16):
                e = g * 16 + i
                tv = t2rows[e, pl.ds(32, 16)]
                dv = d2rows[e, pl.ds(0, 16)]
                ea_e = _vtake(ev, _splat16(i))
                cv = _vtake(dv, _splat16(8))
                a = tv + dv + ea_e * cv
                a = jnp.where(a >= 0.0, a, 0.2 * a)
                w = jnp.exp(a)
                w1 = jnp.where(lane_is0, w, 0.0)
                w_rm[e] = w1
                wb = _vtake(w1, _splat16(0))
                wh2[e, pl.ds(0, 16)] = t2rows[e, pl.ds(0, 16)] * wb
                wh2[e, pl.ds(16, 16)] = t2rows[e, pl.ds(16, 16)] * wb
            return carry2

        lax.fori_loop(0, C2 // 16, grp, 0)

        pltpu.sync_copy(wh2, un_acc.at[dstv], add=True)
        pltpu.sync_copy(w_rm, de_acc.at[dstv], add=True)
        return carry

    lax.fori_loop(0, EPW2 // C2, chunk, 0)
    plsc.subcore_barrier()

    for jj in range(5):
        rr = r0 + jj * 128

        @pl.when(cid == 0)
        def _():
            pltpu.sync_copy(un_acc.at[pl.ds(rr, 128)],
                            un2_a.at[pl.ds(rr, 128)])
            pltpu.sync_copy(de_acc.at[pl.ds(rr, 128)],
                            de2_a.at[pl.ds(rr, 128)])

        @pl.when(cid == 1)
        def _():
            pltpu.sync_copy(un_acc.at[pl.ds(rr, 128)],
                            un2_b.at[pl.ds(rr, 128)])
            pltpu.sync_copy(de_acc.at[pl.ds(rr, 128)],
                            de2_b.at[pl.ds(rr, 128)])


def _sc2(src, dst, ea, t2, d2):
    f32 = jnp.float32
    k = pl.kernel(
        _sc2_body,
        mesh=_MESH,
        compiler_params=pltpu.CompilerParams(use_tc_tiling_on_sc=False),
        out_type=[
            jax.ShapeDtypeStruct((NP, 32), f32),
            jax.ShapeDtypeStruct((NP, 32), f32),
            jax.ShapeDtypeStruct((NP, 16), f32),
            jax.ShapeDtypeStruct((NP, 16), f32),
        ],
        scratch_types=[
            pltpu.VMEM((128, 32), f32),
            pltpu.VMEM((128, 16), f32),
            pltpu.VMEM((C2,), jnp.int32),
            pltpu.VMEM((C2,), jnp.int32),
            pltpu.VMEM((C2,), f32),
            pltpu.VMEM((C2, 48), f32),
            pltpu.VMEM((C2, 16), f32),
            pltpu.VMEM((C2, 32), f32),
            pltpu.VMEM((C2, 16), f32),
            pltpu.VMEM_SHARED((NP, 32), f32),
            pltpu.VMEM_SHARED((NP, 16), f32),
        ],
    )
    return k(src, dst, ea, t2, d2)


def kernel(x, edge_index, edge_attr, batch, W1, a_src1, a_dst1, We1, a_edge1,
           b1, g1, be1, W2, a_src2, a_dst2, We2, a_edge2, b2, g2, be2, Wf, bf):
    f32 = jnp.float32
    src = edge_index[0]
    dst = edge_index[1]
    ea = edge_attr[:, 0]

    W1r = W1.reshape(D_IN, HEADS, HID)
    vsrc = jnp.einsum("dhj,hj->dh", W1r, a_src1)
    vdst = jnp.einsum("dhj,hj->dh", W1r, a_dst1)
    c1 = (We1.reshape(HEADS, HID) * a_edge1).sum(-1)
    wpieces = [W1r[:, 2 * p:2 * p + 2].reshape(D_IN, 64) for p in range(4)]
    vpieces = [jnp.zeros((D_IN, 16), f32).at[:, 0:2].set(
        vsrc[:, 2 * p:2 * p + 2]) for p in range(4)]
    vd = jnp.zeros((D_IN, 16), f32).at[:, 0:8].set(vdst)
    c16 = jnp.zeros((8, 16), f32).at[0, 8:16].set(c1)

    t00, t01, t10, t11, dm = _k1(x, wpieces, vpieces, vd, c16)
    un0, un1, un2, un3, de0, de1, de2, de3 = _sc1(
        src, dst, ea, t00, t01, t10, t11, dm)
    uns = (un0, un1, un2, un3)
    des = (de0, de1, de2, de3)

    bs = [b1[64 * p:64 * p + 64].reshape(1, 64) for p in range(4)]
    s1, s2 = _k3a(uns, des, bs)
    npad = float(NP - N)
    mu = (s1[0] - npad * b1) / N
    var = (s2[0] - npad * b1 * b1) / N - mu * mu
    scale = g1 / jnp.sqrt(var + 1e-5)
    shift = be1 - mu * scale
    scs = [scale[64 * p:64 * p + 64].reshape(1, 64) for p in range(4)]
    shs = [shift[64 * p:64 * p + 64].reshape(1, 64) for p in range(4)]
    w2s = [W2[64 * p:64 * p + 64] for p in range(4)]

    a2 = jnp.zeros((HID, 16), f32).at[:, 0].set(a_src2[0])
    ad2 = jnp.zeros((HID, 16), f32).at[:, 0].set(a_dst2[0])
    c2 = (We2[0] * a_edge2[0]).sum()
    c2r = jnp.zeros((8, 16), f32).at[0, 8].set(c2)

    t2, d2 = _k3b(uns, des, bs, scs, shs, w2s, a2, ad2, c2r)
    un2_a, un2_b, de2_a, de2_b = _sc2(src, dst, ea, t2, d2)

    b2r = b2.reshape(1, HID)
    batch_p = jnp.concatenate(
        [batch, jnp.full((NP - N,), G, jnp.int32)]).reshape(NP, 1)
    t1, t2s = _k5a(un2_a, un2_b, de2_a, de2_b, b2r)
    mu2 = (t1[0] - npad * b2) / N
    var2 = (t2s[0] - npad * b2 * b2) / N - mu2 * mu2
    scale2 = (g2 / jnp.sqrt(var2 + 1e-5)).reshape(1, HID)
    shift2 = (be2 - mu2 * (g2 / jnp.sqrt(var2 + 1e-5))).reshape(1, HID)

    _, _, out = _k5b(un2_a, un2_b, de2_a, de2_b, b2r, scale2, shift2,
                     batch_p, Wf, bf.reshape(1, 32))
    return out


# speedup vs baseline: 23.6635x; 1.3748x over previous
import functools

import jax
import jax.numpy as jnp
from jax import lax
from jax.experimental import pallas as pl
from jax.experimental.pallas import tpu as pltpu
from jax.experimental.pallas import tpu_sc as plsc

N = 10000
E = 320000
D_IN = 128
HID = 32
HEADS = 8
G = 64

NP = 10240
NB = 10
BLK = N // NB
BLKP = NP // NB


def _k1_body(x_ref, w0_ref, w1_ref, w2_ref, w3_ref, vs_ref, vd_ref,
             t0_ref, t1_ref, t2_ref, t3_ref, sa_ref, da_ref):
    xb = x_ref[...]
    f32 = jnp.float32
    for w_ref, t_ref in ((w0_ref, t0_ref), (w1_ref, t1_ref),
                         (w2_ref, t2_ref), (w3_ref, t3_ref)):
        t_ref[...] = jnp.dot(xb, w_ref[...], preferred_element_type=f32)
    sa_ref[...] = jnp.dot(xb, vs_ref[...], preferred_element_type=f32)
    da_ref[...] = jnp.dot(xb, vd_ref[...], preferred_element_type=f32)


def _k1(x, wpieces, vs, vd):
    f32 = jnp.float32
    wspec = pl.BlockSpec((128, 64), lambda i: (0, 0))
    vspec = pl.BlockSpec((128, 8), lambda i: (0, 0))
    tspec = pl.BlockSpec((BLK, 64), lambda i: (i, 0))
    aspec = pl.BlockSpec((BLK, 8), lambda i: (i, 0))
    tshape = jax.ShapeDtypeStruct((N, 64), f32)
    ashape = jax.ShapeDtypeStruct((N, 8), f32)
    return pl.pallas_call(
        _k1_body,
        grid=(NB,),
        in_specs=[pl.BlockSpec((BLK, 128), lambda i: (i, 0)),
                  wspec, wspec, wspec, wspec, vspec, vspec],
        out_specs=[tspec, tspec, tspec, tspec, aspec, aspec],
        out_shape=[tshape, tshape, tshape, tshape, ashape, ashape],
    )(x, *wpieces, vs, vd)


def _bn_stats_body(u0, u1, u2, u3, b0, b1r, b2r_, b3, s1_ref, s2_ref):
    i = pl.program_id(0)

    @pl.when(i == 0)
    def _():
        s1_ref[...] = jnp.zeros_like(s1_ref)
        s2_ref[...] = jnp.zeros_like(s2_ref)

    for p, (un_ref, b_ref) in enumerate(
            ((u0, b0), (u1, b1r), (u2, b2r_), (u3, b3))):
        for k in range(2):
            sl = slice(k * 32, (k + 1) * 32)
            v = (un_ref[:, sl] / (un_ref[:, 64 + k:65 + k] + 1e-16)
                 + b_ref[0:1, sl])
            osl = slice(p * 64 + k * 32, p * 64 + (k + 1) * 32)
            s1_ref[0:1, osl] += jnp.sum(v, axis=0, keepdims=True)
            s2_ref[0:1, osl] += jnp.sum(v * v, axis=0, keepdims=True)


def _k3a(uns, bs):
    f32 = jnp.float32
    uspec = pl.BlockSpec((BLKP, 80), lambda i: (i, 0))
    bspec = pl.BlockSpec((1, 64), lambda i: (0, 0))
    return pl.pallas_call(
        _bn_stats_body,
        grid=(NB,),
        in_specs=[uspec] * 4 + [bspec] * 4,
        out_specs=[
            pl.BlockSpec((8, 256), lambda i: (0, 0)),
            pl.BlockSpec((8, 256), lambda i: (0, 0)),
        ],
        out_shape=[
            jax.ShapeDtypeStruct((8, 256), f32),
            jax.ShapeDtypeStruct((8, 256), f32),
        ],
    )(*uns, *bs)


def _k3b_body(u0, u1, u2, u3, b0, b1r, b2r_, b3,
              sc0, sc1, sc2, sc3, sh0, sh1, sh2, sh3, w0, w1, w2_, w3,
              a2_ref, ad2_ref, t2_ref, sa2_ref, da2_ref):
    f32 = jnp.float32
    h2 = jnp.zeros((BLKP, HID), f32)
    for un_ref, b_ref, sc_ref, sh_ref, w2_ref in (
            (u0, b0, sc0, sh0, w0), (u1, b1r, sc1, sh1, w1),
            (u2, b2r_, sc2, sh2, w2_), (u3, b3, sc3, sh3, w3)):
        for k in range(2):
            sl = slice(k * 32, (k + 1) * 32)
            v = (un_ref[:, sl] / (un_ref[:, 64 + k:65 + k] + 1e-16)
                 + b_ref[0:1, sl])
            v = v * sc_ref[0:1, sl] + sh_ref[0:1, sl]
            v = jnp.maximum(v, 0.0)
            h2 = h2 + jnp.dot(v, w2_ref[sl, :], preferred_element_type=f32)
    t2_ref[...] = h2
    sa2_ref[...] = jnp.dot(h2, a2_ref[...], preferred_element_type=f32)
    da2_ref[...] = jnp.dot(h2, ad2_ref[...], preferred_element_type=f32)


def _k3b(uns, bs, scs, shs, w2s, a2, ad2):
    f32 = jnp.float32
    uspec = pl.BlockSpec((BLKP, 80), lambda i: (i, 0))
    bspec = pl.BlockSpec((1, 64), lambda i: (0, 0))
    wspec = pl.BlockSpec((64, 32), lambda i: (0, 0))
    return pl.pallas_call(
        _k3b_body,
        grid=(NB,),
        in_specs=[uspec] * 4 + [bspec] * 12 + [wspec] * 4 + [
            pl.BlockSpec((32, 8), lambda i: (0, 0)),
            pl.BlockSpec((32, 8), lambda i: (0, 0)),
        ],
        out_specs=[
            pl.BlockSpec((BLKP, 32), lambda i: (i, 0)),
            pl.BlockSpec((BLKP, 8), lambda i: (i, 0)),
            pl.BlockSpec((BLKP, 8), lambda i: (i, 0)),
        ],
        out_shape=[
            jax.ShapeDtypeStruct((NP, 32), f32),
            jax.ShapeDtypeStruct((NP, 8), f32),
            jax.ShapeDtypeStruct((NP, 8), f32),
        ],
    )(*uns, *bs, *scs, *shs, *w2s, a2, ad2)


def _bn2_stats_body(una_ref, unb_ref, b2_ref, s1_ref, s2_ref):
    i = pl.program_id(0)

    @pl.when(i == 0)
    def _():
        s1_ref[...] = jnp.zeros_like(s1_ref)
        s2_ref[...] = jnp.zeros_like(s2_ref)

    de = una_ref[:, 32:33] + unb_ref[:, 32:33] + 1e-16
    v = (una_ref[:, 0:32] + unb_ref[:, 0:32]) / de + b2_ref[0:1, :]
    s1_ref[0:1, :] += jnp.sum(v, axis=0, keepdims=True)
    s2_ref[0:1, :] += jnp.sum(v * v, axis=0, keepdims=True)


def _k5a(un2_a, un2_b, b2r):
    f32 = jnp.float32
    return pl.pallas_call(
        _bn2_stats_body,
        grid=(NB,),
        in_specs=[
            pl.BlockSpec((BLKP, 48), lambda i: (i, 0)),
            pl.BlockSpec((BLKP, 48), lambda i: (i, 0)),
            pl.BlockSpec((1, 32), lambda i: (0, 0)),
        ],
        out_specs=[
            pl.BlockSpec((8, 32), lambda i: (0, 0)),
            pl.BlockSpec((8, 32), lambda i: (0, 0)),
        ],
        out_shape=[
            jax.ShapeDtypeStruct((8, 32), f32),
            jax.ShapeDtypeStruct((8, 32), f32),
        ],
    )(un2_a, un2_b, b2r)


def _pool_body(una_ref, unb_ref, b2_ref, sc2_ref, sh2_ref,
               batch_ref, wf_ref, bf_ref, p_ref, cnt_ref, out_ref):
    i = pl.program_id(0)
    f32 = jnp.float32

    @pl.when(i == 0)
    def _():
        p_ref[...] = jnp.zeros_like(p_ref)
        cnt_ref[...] = jnp.zeros_like(cnt_ref)

    de = una_ref[:, 32:33] + unb_ref[:, 32:33] + 1e-16
    v = (una_ref[:, 0:32] + unb_ref[:, 0:32]) / de + b2_ref[0:1, :]
    h = jnp.maximum(v * sc2_ref[0:1, :] + sh2_ref[0:1, :], 0.0)
    oh = (batch_ref[...] == lax.broadcasted_iota(jnp.int32, (BLKP, G), 1)
          ).astype(f32)
    dn = (((0,), (0,)), ((), ()))
    p_ref[...] += lax.dot_general(oh, h, dn, preferred_element_type=f32)
    cnt_ref[...] += lax.dot_general(oh, jnp.ones((BLKP, 8), f32), dn,
                                    preferred_element_type=f32)

    @pl.when(i == NB - 1)
    def _():
        pooled = p_ref[...] / jnp.maximum(cnt_ref[:, 0:1], 1.0)
        out_ref[...] = (jnp.dot(pooled, wf_ref[...],
                                preferred_element_type=f32) + bf_ref[0:1, :])


def _k5b(un2_a, un2_b, b2r, sc2, sh2, batch2d, wf, bfr):
    f32 = jnp.float32
    row32 = pl.BlockSpec((1, 32), lambda i: (0, 0))
    return pl.pallas_call(
        _pool_body,
        grid=(NB,),
        in_specs=[
            pl.BlockSpec((BLKP, 48), lambda i: (i, 0)),
            pl.BlockSpec((BLKP, 48), lambda i: (i, 0)),
            row32, row32, row32,
            pl.BlockSpec((BLKP, 1), lambda i: (i, 0)),
            pl.BlockSpec((32, 32), lambda i: (0, 0)),
            row32,
        ],
        out_specs=[
            pl.BlockSpec((G, 32), lambda i: (0, 0)),
            pl.BlockSpec((G, 8), lambda i: (0, 0)),
            pl.BlockSpec((G, 32), lambda i: (0, 0)),
        ],
        out_shape=[
            jax.ShapeDtypeStruct((G, 32), f32),
            jax.ShapeDtypeStruct((G, 8), f32),
            jax.ShapeDtypeStruct((G, 32), f32),
        ],
    )(un2_a, un2_b, b2r, sc2, sh2, batch2d, wf, bfr)


_MESH = plsc.VectorSubcoreMesh(core_axis_name="c", subcore_axis_name="s")


def _vtake(x, idx):
    return jnp.take_along_axis(x, idx, axis=0, mode="promise_in_bounds")


def _splat16(i):
    return jnp.full((16,), i, jnp.int32)

EPT1 = E // 16
C1 = 400
ROWS_PT = NP // 16
EPW2 = E // 32
C2 = 400


def _sc1_body(src, dst, ea, h0, h1, h2, h3,
              sa0, sa1, sa2, sa3, sa4, sa5, sa6, sa7,
              da0, da1, da2, da3, da4, da5, da6, da7, c16,
              un0, un1, un2, un3,
              zb, cbuf, srcv, dstv, eav, asr0, asr1, ads0, ads1, trows, wh,
              un_acc):
    f32 = jnp.float32
    z16 = jnp.zeros((16,), f32)
    cid = lax.axis_index("c")
    sid = lax.axis_index("s")
    iota16 = lax.iota(jnp.int32, 16)
    lane0 = iota16 == 0
    lane1 = iota16 == 1
    zi = jnp.zeros((16,), jnp.int32)

    pltpu.sync_copy(c16, cbuf)
    cv16 = cbuf[pl.ds(0, 16)]

    def zrow(i, carry):
        for j in range(5):
            zb[i, pl.ds(j * 16, 16)] = z16
        return carry

    lax.fori_loop(0, 64, zrow, 0)

    r0 = sid * ROWS_PT
    ebase = sid * EPT1
    sas = (sa0, sa1, sa2, sa3, sa4, sa5, sa6, sa7)
    das = (da0, da1, da2, da3, da4, da5, da6, da7)

    for q, (hc0, hc1, uo0, uo1) in enumerate(
            ((h0, h2, un0, un2), (h1, h3, un1, un3))):
        hco = cid * 4 + q * 2
        ch0 = _vtake(cv16, zi + hco)
        ch1 = _vtake(cv16, zi + (hco + 1))

        for jj in range(10):
            pltpu.sync_copy(zb, un_acc.at[pl.ds(r0 + jj * 64, 64)])
        plsc.subcore_barrier()

        def chunk(j, carry):
            base = ebase + j * C1
            pltpu.sync_copy(src.at[pl.ds(base, C1)], srcv)
            pltpu.sync_copy(dst.at[pl.ds(base, C1)], dstv)
            pltpu.sync_copy(ea.at[pl.ds(base, C1)], eav)

            @pl.when(cid == 0)
            def _():
                pltpu.sync_copy(hc0.at[srcv], trows)
                pltpu.sync_copy(sas[2 * q].at[srcv], asr0)
                pltpu.sync_copy(sas[2 * q + 1].at[srcv], asr1)
                pltpu.sync_copy(das[2 * q].at[dstv], ads0)
                pltpu.sync_copy(das[2 * q + 1].at[dstv], ads1)

            @pl.when(cid == 1)
            def _():
                pltpu.sync_copy(hc1.at[srcv], trows)
                pltpu.sync_copy(sas[4 + 2 * q].at[srcv], asr0)
                pltpu.sync_copy(sas[4 + 2 * q + 1].at[srcv], asr1)
                pltpu.sync_copy(das[4 + 2 * q].at[dstv], ads0)
                pltpu.sync_copy(das[4 + 2 * q + 1].at[dstv], ads1)

            def grp(g, carry2):
                sl16 = pl.ds(g * 16, 16)
                ev = eav[sl16]
                a0 = asr0[sl16] + ads0[sl16] + ev * ch0
                a1 = asr1[sl16] + ads1[sl16] + ev * ch1
                a0 = jnp.where(a0 >= 0.0, a0, 0.2 * a0)
                a1 = jnp.where(a1 >= 0.0, a1, 0.2 * a1)
                w0 = jnp.exp(a0)
                w1 = jnp.exp(a1)
                for i in range(16):
                    e = g * 16 + i
                    si = _splat16(i)
                    wb0 = _vtake(w0, si)
                    wb1 = _vtake(w1, si)
                    wh[e, pl.ds(64, 16)] = jnp.where(
                        lane0, wb0, jnp.where(lane1, wb1, 0.0))
                    for r in range(4):
                        sl = pl.ds(r * 16, 16)
                        wh[e, sl] = trows[e, sl] * (wb0 if r < 2 else wb1)
                return carry2

            lax.fori_loop(0, C1 // 16, grp, 0)

            pltpu.sync_copy(wh, un_acc.at[dstv], add=True)
            return carry

        lax.fori_loop(0, EPT1 // C1, chunk, 0)
        plsc.subcore_barrier()

        for jj in range(5):
            rr = r0 + jj * 128

            @pl.when(cid == 0)
            def _():
                pltpu.sync_copy(un_acc.at[pl.ds(rr, 128)],
                                uo0.at[pl.ds(rr, 128)])

            @pl.when(cid == 1)
            def _():
                pltpu.sync_copy(un_acc.at[pl.ds(rr, 128)],
                                uo1.at[pl.ds(rr, 128)])


def _sc1(src, dst, ea, hp, sa_cols, da_cols, c16):
    f32 = jnp.float32
    k = pl.kernel(
        _sc1_body,
        mesh=_MESH,
        compiler_params=pltpu.CompilerParams(use_tc_tiling_on_sc=False),
        out_type=[jax.ShapeDtypeStruct((NP, 80), f32)] * 4,
        scratch_types=[
            pltpu.VMEM((64, 80), f32),
            pltpu.VMEM((16,), f32),
            pltpu.VMEM((C1,), jnp.int32),
            pltpu.VMEM((C1,), jnp.int32),
            pltpu.VMEM((C1,), f32),
            pltpu.VMEM((C1,), f32),
            pltpu.VMEM((C1,), f32),
            pltpu.VMEM((C1,), f32),
            pltpu.VMEM((C1,), f32),
            pltpu.VMEM((C1, 64), f32),
            pltpu.VMEM((C1, 80), f32),
            pltpu.VMEM_SHARED((NP, 80), f32),
        ],
    )
    return k(src, dst, ea, *hp, *sa_cols, *da_cols, c16)


def _sc2_body(src, dst, ea, t2, sa2, da2, c2v,
              un2_a, un2_b,
              zb, cbuf, srcv, dstv, eav, asrv, adsv, t2rows, wh2,
              un_acc):
    f32 = jnp.float32
    z16 = jnp.zeros((16,), f32)
    cid = lax.axis_index("c")
    sid = lax.axis_index("s")
    wid = cid * 16 + sid
    iota16 = lax.iota(jnp.int32, 16)
    lane0 = iota16 == 0
    zi = jnp.zeros((16,), jnp.int32)

    pltpu.sync_copy(c2v, cbuf)
    ch = _vtake(cbuf[pl.ds(0, 16)], zi)

    def zrow(i, carry):
        for j in range(3):
            zb[i, pl.ds(j * 16, 16)] = z16
        return carry

    lax.fori_loop(0, 64, zrow, 0)

    r0 = sid * ROWS_PT
    for jj in range(10):
        pltpu.sync_copy(zb, un_acc.at[pl.ds(r0 + jj * 64, 64)])
    plsc.subcore_barrier()

    ebase = wid * EPW2

    def chunk(j, carry):
        base = ebase + j * C2
        pltpu.sync_copy(src.at[pl.ds(base, C2)], srcv)
        pltpu.sync_copy(dst.at[pl.ds(base, C2)], dstv)
        pltpu.sync_copy(ea.at[pl.ds(base, C2)], eav)
        pltpu.sync_copy(t2.at[srcv], t2rows)
        pltpu.sync_copy(sa2.at[srcv], asrv)
        pltpu.sync_copy(da2.at[dstv], adsv)

        def grp(g, carry2):
            sl16 = pl.ds(g * 16, 16)
            a = asrv[sl16] + adsv[sl16] + eav[sl16] * ch
            a = jnp.where(a >= 0.0, a, 0.2 * a)
            w = jnp.exp(a)
            for i in range(16):
                e = g * 16 + i
                wb = _vtake(w, _splat16(i))
                wh2[e, pl.ds(32, 16)] = jnp.where(lane0, wb, 0.0)
                wh2[e, pl.ds(0, 16)] = t2rows[e, pl.ds(0, 16)] * wb
                wh2[e, pl.ds(16, 16)] = t2rows[e, pl.ds(16, 16)] * wb
            return carry2

        lax.fori_loop(0, C2 // 16, grp, 0)

        pltpu.sync_copy(wh2, un_acc.at[dstv], add=True)
        return carry

    lax.fori_loop(0, EPW2 // C2, chunk, 0)
    plsc.subcore_barrier()

    for jj in range(5):
        rr = r0 + jj * 128

        @pl.when(cid == 0)
        def _():
            pltpu.sync_copy(un_acc.at[pl.ds(rr, 128)],
                            un2_a.at[pl.ds(rr, 128)])

        @pl.when(cid == 1)
        def _():
            pltpu.sync_copy(un_acc.at[pl.ds(rr, 128)],
                            un2_b.at[pl.ds(rr, 128)])


def _sc2(src, dst, ea, t2, sa2, da2, c2v):
    f32 = jnp.float32
    k = pl.kernel(
        _sc2_body,
        mesh=_MESH,
        compiler_params=pltpu.CompilerParams(use_tc_tiling_on_sc=False),
        out_type=[jax.ShapeDtypeStruct((NP, 48), f32)] * 2,
        scratch_types=[
            pltpu.VMEM((64, 48), f32),
            pltpu.VMEM((16,), f32),
            pltpu.VMEM((C2,), jnp.int32),
            pltpu.VMEM((C2,), jnp.int32),
            pltpu.VMEM((C2,), f32),
            pltpu.VMEM((C2,), f32),
            pltpu.VMEM((C2,), f32),
            pltpu.VMEM((C2, 32), f32),
            pltpu.VMEM((C2, 48), f32),
            pltpu.VMEM_SHARED((NP, 48), f32),
        ],
    )
    return k(src, dst, ea, t2, sa2, da2, c2v)


def kernel(x, edge_index, edge_attr, batch, W1, a_src1, a_dst1, We1, a_edge1,
           b1, g1, be1, W2, a_src2, a_dst2, We2, a_edge2, b2, g2, be2, Wf, bf):
    f32 = jnp.float32
    src = edge_index[0]
    dst = edge_index[1]
    ea = edge_attr[:, 0]

    W1r = W1.reshape(D_IN, HEADS, HID)
    vsrc = jnp.einsum("dhj,hj->dh", W1r, a_src1)
    vdst = jnp.einsum("dhj,hj->dh", W1r, a_dst1)
    c1 = (We1.reshape(HEADS, HID) * a_edge1).sum(-1)
    wpieces = [W1r[:, 2 * p:2 * p + 2].reshape(D_IN, 64) for p in range(4)]
    c16 = jnp.zeros((16,), f32).at[0:8].set(c1)

    hp0, hp1, hp2, hp3, sa, da = _k1(x, wpieces, vsrc, vdst)
    sa_cols = [sa[:, h] for h in range(HEADS)]
    da_cols = [da[:, h] for h in range(HEADS)]
    uns = _sc1(src, dst, ea, (hp0, hp1, hp2, hp3), sa_cols, da_cols, c16)

    bs = [b1[64 * p:64 * p + 64].reshape(1, 64) for p in range(4)]
    s1, s2 = _k3a(uns, bs)
    npad = float(NP - N)
    mu = (s1[0] - npad * b1) / N
    var = (s2[0] - npad * b1 * b1) / N - mu * mu
    scale = g1 / jnp.sqrt(var + 1e-5)
    shift = be1 - mu * scale
    scs = [scale[64 * p:64 * p + 64].reshape(1, 64) for p in range(4)]
    shs = [shift[64 * p:64 * p + 64].reshape(1, 64) for p in range(4)]
    w2s = [W2[64 * p:64 * p + 64] for p in range(4)]

    a2 = jnp.zeros((HID, 8), f32).at[:, 0].set(a_src2[0])
    ad2 = jnp.zeros((HID, 8), f32).at[:, 0].set(a_dst2[0])
    c2 = (We2[0] * a_edge2[0]).sum()
    c2v = jnp.zeros((16,), f32).at[0].set(c2)

    t2, sa2, da2 = _k3b(uns, bs, scs, shs, w2s, a2, ad2)
    un2_a, un2_b = _sc2(src, dst, ea, t2, sa2[:, 0], da2[:, 0], c2v)

    b2r = b2.reshape(1, HID)
    batch_p = jnp.concatenate(
        [batch, jnp.full((NP - N,), G, jnp.int32)]).reshape(NP, 1)
    t1, t2s = _k5a(un2_a, un2_b, b2r)
    mu2 = (t1[0] - npad * b2) / N
    var2 = (t2s[0] - npad * b2 * b2) / N - mu2 * mu2
    scale2 = (g2 / jnp.sqrt(var2 + 1e-5)).reshape(1, HID)
    shift2 = (be2 - mu2 * (g2 / jnp.sqrt(var2 + 1e-5))).reshape(1, HID)

    _, _, out = _k5b(un2_a, un2_b, b2r, scale2, shift2,
                     batch_p, Wf, bf.reshape(1, 32))
    return out


# speedup vs baseline: 25.8646x; 1.0930x over previous
import functools

import jax
import jax.numpy as jnp
from jax import lax
from jax.experimental import pallas as pl
from jax.experimental.pallas import tpu as pltpu
from jax.experimental.pallas import tpu_sc as plsc

N = 10000
E = 320000
D_IN = 128
HID = 32
HEADS = 8
G = 64

NP = 10240
NB = 10
BLK = N // NB
BLKP = NP // NB


def _k1_body(x_ref, w0_ref, w1_ref, w2_ref, w3_ref, vs_ref, vd_ref,
             t0_ref, t1_ref, t2_ref, t3_ref, sa_ref, da_ref):
    xb = x_ref[...]
    f32 = jnp.float32
    for w_ref, t_ref in ((w0_ref, t0_ref), (w1_ref, t1_ref),
                         (w2_ref, t2_ref), (w3_ref, t3_ref)):
        t_ref[...] = jnp.dot(xb, w_ref[...], preferred_element_type=f32)
    sa_ref[...] = jnp.dot(xb, vs_ref[...], preferred_element_type=f32)
    da_ref[...] = jnp.dot(xb, vd_ref[...], preferred_element_type=f32)


def _k1(x, wpieces, vs, vd):
    f32 = jnp.float32
    wspec = pl.BlockSpec((128, 64), lambda i: (0, 0))
    vspec = pl.BlockSpec((128, 8), lambda i: (0, 0))
    tspec = pl.BlockSpec((BLK, 64), lambda i: (i, 0))
    aspec = pl.BlockSpec((BLK, 8), lambda i: (i, 0))
    tshape = jax.ShapeDtypeStruct((N, 64), f32)
    ashape = jax.ShapeDtypeStruct((N, 8), f32)
    return pl.pallas_call(
        _k1_body,
        grid=(NB,),
        in_specs=[pl.BlockSpec((BLK, 128), lambda i: (i, 0)),
                  wspec, wspec, wspec, wspec, vspec, vspec],
        out_specs=[tspec, tspec, tspec, tspec, aspec, aspec],
        out_shape=[tshape, tshape, tshape, tshape, ashape, ashape],
    )(x, *wpieces, vs, vd)


def _bn_stats_body(u0, u1, u2, u3, b0, b1r, b2r_, b3, s1_ref, s2_ref):
    i = pl.program_id(0)

    @pl.when(i == 0)
    def _():
        s1_ref[...] = jnp.zeros_like(s1_ref)
        s2_ref[...] = jnp.zeros_like(s2_ref)

    for p, (un_ref, b_ref) in enumerate(
            ((u0, b0), (u1, b1r), (u2, b2r_), (u3, b3))):
        for k in range(2):
            sl = slice(k * 32, (k + 1) * 32)
            v = (un_ref[:, sl] / (un_ref[:, 64 + k:65 + k] + 1e-16)
                 + b_ref[0:1, sl])
            osl = slice(p * 64 + k * 32, p * 64 + (k + 1) * 32)
            s1_ref[0:1, osl] += jnp.sum(v, axis=0, keepdims=True)
            s2_ref[0:1, osl] += jnp.sum(v * v, axis=0, keepdims=True)


def _k3a(uns, bs):
    f32 = jnp.float32
    uspec = pl.BlockSpec((BLKP, 80), lambda i: (i, 0))
    bspec = pl.BlockSpec((1, 64), lambda i: (0, 0))
    return pl.pallas_call(
        _bn_stats_body,
        grid=(NB,),
        in_specs=[uspec] * 4 + [bspec] * 4,
        out_specs=[
            pl.BlockSpec((8, 256), lambda i: (0, 0)),
            pl.BlockSpec((8, 256), lambda i: (0, 0)),
        ],
        out_shape=[
            jax.ShapeDtypeStruct((8, 256), f32),
            jax.ShapeDtypeStruct((8, 256), f32),
        ],
    )(*uns, *bs)


def _k3b_body(u0, u1, u2, u3, b0, b1r, b2r_, b3,
              sc0, sc1, sc2, sc3, sh0, sh1, sh2, sh3, w0, w1, w2_, w3,
              a2_ref, ad2_ref, t2_ref, sa2_ref, da2_ref):
    f32 = jnp.float32
    h2 = jnp.zeros((BLKP, HID), f32)
    for un_ref, b_ref, sc_ref, sh_ref, w2_ref in (
            (u0, b0, sc0, sh0, w0), (u1, b1r, sc1, sh1, w1),
            (u2, b2r_, sc2, sh2, w2_), (u3, b3, sc3, sh3, w3)):
        for k in range(2):
            sl = slice(k * 32, (k + 1) * 32)
            v = (un_ref[:, sl] / (un_ref[:, 64 + k:65 + k] + 1e-16)
                 + b_ref[0:1, sl])
            v = v * sc_ref[0:1, sl] + sh_ref[0:1, sl]
            v = jnp.maximum(v, 0.0)
            h2 = h2 + jnp.dot(v, w2_ref[sl, :], preferred_element_type=f32)
    t2_ref[...] = h2
    sa2_ref[...] = jnp.dot(h2, a2_ref[...], preferred_element_type=f32)
    da2_ref[...] = jnp.dot(h2, ad2_ref[...], preferred_element_type=f32)


def _k3b(uns, bs, scs, shs, w2s, a2, ad2):
    f32 = jnp.float32
    uspec = pl.BlockSpec((BLKP, 80), lambda i: (i, 0))
    bspec = pl.BlockSpec((1, 64), lambda i: (0, 0))
    wspec = pl.BlockSpec((64, 32), lambda i: (0, 0))
    return pl.pallas_call(
        _k3b_body,
        grid=(NB,),
        in_specs=[uspec] * 4 + [bspec] * 12 + [wspec] * 4 + [
            pl.BlockSpec((32, 8), lambda i: (0, 0)),
            pl.BlockSpec((32, 8), lambda i: (0, 0)),
        ],
        out_specs=[
            pl.BlockSpec((BLKP, 32), lambda i: (i, 0)),
            pl.BlockSpec((BLKP, 8), lambda i: (i, 0)),
            pl.BlockSpec((BLKP, 8), lambda i: (i, 0)),
        ],
        out_shape=[
            jax.ShapeDtypeStruct((NP, 32), f32),
            jax.ShapeDtypeStruct((NP, 8), f32),
            jax.ShapeDtypeStruct((NP, 8), f32),
        ],
    )(*uns, *bs, *scs, *shs, *w2s, a2, ad2)


def _bn2_stats_body(una_ref, unb_ref, b2_ref, s1_ref, s2_ref):
    i = pl.program_id(0)

    @pl.when(i == 0)
    def _():
        s1_ref[...] = jnp.zeros_like(s1_ref)
        s2_ref[...] = jnp.zeros_like(s2_ref)

    de = una_ref[:, 32:33] + unb_ref[:, 32:33] + 1e-16
    v = (una_ref[:, 0:32] + unb_ref[:, 0:32]) / de + b2_ref[0:1, :]
    s1_ref[0:1, :] += jnp.sum(v, axis=0, keepdims=True)
    s2_ref[0:1, :] += jnp.sum(v * v, axis=0, keepdims=True)


def _k5a(un2_a, un2_b, b2r):
    f32 = jnp.float32
    return pl.pallas_call(
        _bn2_stats_body,
        grid=(NB,),
        in_specs=[
            pl.BlockSpec((BLKP, 48), lambda i: (i, 0)),
            pl.BlockSpec((BLKP, 48), lambda i: (i, 0)),
            pl.BlockSpec((1, 32), lambda i: (0, 0)),
        ],
        out_specs=[
            pl.BlockSpec((8, 32), lambda i: (0, 0)),
            pl.BlockSpec((8, 32), lambda i: (0, 0)),
        ],
        out_shape=[
            jax.ShapeDtypeStruct((8, 32), f32),
            jax.ShapeDtypeStruct((8, 32), f32),
        ],
    )(un2_a, un2_b, b2r)


def _pool_body(una_ref, unb_ref, b2_ref, sc2_ref, sh2_ref,
               batch_ref, wf_ref, bf_ref, p_ref, cnt_ref, out_ref):
    i = pl.program_id(0)
    f32 = jnp.float32

    @pl.when(i == 0)
    def _():
        p_ref[...] = jnp.zeros_like(p_ref)
        cnt_ref[...] = jnp.zeros_like(cnt_ref)

    de = una_ref[:, 32:33] + unb_ref[:, 32:33] + 1e-16
    v = (una_ref[:, 0:32] + unb_ref[:, 0:32]) / de + b2_ref[0:1, :]
    h = jnp.maximum(v * sc2_ref[0:1, :] + sh2_ref[0:1, :], 0.0)
    oh = (batch_ref[...] == lax.broadcasted_iota(jnp.int32, (BLKP, G), 1)
          ).astype(f32)
    dn = (((0,), (0,)), ((), ()))
    p_ref[...] += lax.dot_general(oh, h, dn, preferred_element_type=f32)
    cnt_ref[...] += lax.dot_general(oh, jnp.ones((BLKP, 8), f32), dn,
                                    preferred_element_type=f32)

    @pl.when(i == NB - 1)
    def _():
        pooled = p_ref[...] / jnp.maximum(cnt_ref[:, 0:1], 1.0)
        out_ref[...] = (jnp.dot(pooled, wf_ref[...],
                                preferred_element_type=f32) + bf_ref[0:1, :])


def _k5b(un2_a, un2_b, b2r, sc2, sh2, batch2d, wf, bfr):
    f32 = jnp.float32
    row32 = pl.BlockSpec((1, 32), lambda i: (0, 0))
    return pl.pallas_call(
        _pool_body,
        grid=(NB,),
        in_specs=[
            pl.BlockSpec((BLKP, 48), lambda i: (i, 0)),
            pl.BlockSpec((BLKP, 48), lambda i: (i, 0)),
            row32, row32, row32,
            pl.BlockSpec((BLKP, 1), lambda i: (i, 0)),
            pl.BlockSpec((32, 32), lambda i: (0, 0)),
            row32,
        ],
        out_specs=[
            pl.BlockSpec((G, 32), lambda i: (0, 0)),
            pl.BlockSpec((G, 8), lambda i: (0, 0)),
            pl.BlockSpec((G, 32), lambda i: (0, 0)),
        ],
        out_shape=[
            jax.ShapeDtypeStruct((G, 32), f32),
            jax.ShapeDtypeStruct((G, 8), f32),
            jax.ShapeDtypeStruct((G, 32), f32),
        ],
    )(un2_a, un2_b, b2r, sc2, sh2, batch2d, wf, bfr)


_MESH = plsc.VectorSubcoreMesh(core_axis_name="c", subcore_axis_name="s")


def _vtake(x, idx):
    return jnp.take_along_axis(x, idx, axis=0, mode="promise_in_bounds")


def _splat16(i):
    return jnp.full((16,), i, jnp.int32)

EPT1 = E // 16
C1 = 160
ROWS_PT = NP // 16
EPW2 = E // 32
C2 = 400


def _sc1_body(src, dst, ea, h0, h1, h2, h3,
              sa0, sa1, sa2, sa3, sa4, sa5, sa6, sa7,
              da0, da1, da2, da3, da4, da5, da6, da7, c16,
              un0, un1, un2, un3,
              zb, cbuf,
              srcv_0, dstv_0, eav_0, asr0_0, asr1_0, ads0_0, ads1_0, trows_0,
              srcv_1, dstv_1, eav_1, asr0_1, asr1_1, ads0_1, ads1_1, trows_1,
              wh, un_acc, sem0, sem1):
    f32 = jnp.float32
    z16 = jnp.zeros((16,), f32)
    cid = lax.axis_index("c")
    sid = lax.axis_index("s")
    iota16 = lax.iota(jnp.int32, 16)
    lane0 = iota16 == 0
    lane1 = iota16 == 1
    zi = jnp.zeros((16,), jnp.int32)

    pltpu.sync_copy(c16, cbuf)
    cv16 = cbuf[pl.ds(0, 16)]

    def zrow(i, carry):
        for j in range(5):
            zb[i, pl.ds(j * 16, 16)] = z16
        return carry

    lax.fori_loop(0, 64, zrow, 0)

    r0 = sid * ROWS_PT
    ebase = sid * EPT1
    sas = (sa0, sa1, sa2, sa3, sa4, sa5, sa6, sa7)
    das = (da0, da1, da2, da3, da4, da5, da6, da7)
    bufs = ((srcv_0, dstv_0, eav_0, asr0_0, asr1_0, ads0_0, ads1_0,
             trows_0, sem0),
            (srcv_1, dstv_1, eav_1, asr0_1, asr1_1, ads0_1, ads1_1,
             trows_1, sem1))
    NCH = EPT1 // C1

    for q, (hc0, hc1, uo0, uo1) in enumerate(
            ((h0, h2, un0, un2), (h1, h3, un1, un3))):
        hco = cid * 4 + q * 2
        ch0 = _vtake(cv16, zi + hco)
        ch1 = _vtake(cv16, zi + (hco + 1))

        def load_idx(jj, bu):
            base = ebase + jj * C1
            pltpu.sync_copy(src.at[pl.ds(base, C1)], bu[0])
            pltpu.sync_copy(dst.at[pl.ds(base, C1)], bu[1])
            pltpu.sync_copy(ea.at[pl.ds(base, C1)], bu[2])

        def fire_gathers(hc, s0t, s1t, d0t, d1t, bu):
            sem = bu[8]
            return [pltpu.async_copy(hc.at[bu[0]], bu[7], sem),
                    pltpu.async_copy(s0t.at[bu[0]], bu[3], sem),
                    pltpu.async_copy(s1t.at[bu[0]], bu[4], sem),
                    pltpu.async_copy(d0t.at[bu[1]], bu[5], sem),
                    pltpu.async_copy(d1t.at[bu[1]], bu[6], sem)]

        def compute_scatter(bu):
            eav_b, asr0_b, asr1_b = bu[2], bu[3], bu[4]
            ads0_b, ads1_b, trows_b = bu[5], bu[6], bu[7]

            def grp(g, carry2):
                sl16 = pl.ds(g * 16, 16)
                ev = eav_b[sl16]
                a0 = asr0_b[sl16] + ads0_b[sl16] + ev * ch0
                a1 = asr1_b[sl16] + ads1_b[sl16] + ev * ch1
                a0 = jnp.where(a0 >= 0.0, a0, 0.2 * a0)
                a1 = jnp.where(a1 >= 0.0, a1, 0.2 * a1)
                w0 = jnp.exp(a0)
                w1 = jnp.exp(a1)
                for i in range(16):
                    e = g * 16 + i
                    si = _splat16(i)
                    wb0 = _vtake(w0, si)
                    wb1 = _vtake(w1, si)
                    wh[e, pl.ds(64, 16)] = jnp.where(
                        lane0, wb0, jnp.where(lane1, wb1, 0.0))
                    for r in range(4):
                        sl = pl.ds(r * 16, 16)
                        wh[e, sl] = trows_b[e, sl] * (wb0 if r < 2 else wb1)
                return carry2

            lax.fori_loop(0, C1 // 16, grp, 0)
            pltpu.sync_copy(wh, un_acc.at[bu[1]], add=True)

        for jj in range(10):
            pltpu.sync_copy(zb, un_acc.at[pl.ds(r0 + jj * 64, 64)])
        plsc.subcore_barrier()

        variants = (
            (0, hc0, sas[2 * q], sas[2 * q + 1],
             das[2 * q], das[2 * q + 1]),
            (1, hc1, sas[4 + 2 * q], sas[4 + 2 * q + 1],
             das[4 + 2 * q], das[4 + 2 * q + 1]),
        )

        def pair(jp, carry):
            for cv, hc, s0t, s1t, d0t, d1t in variants:

                @pl.when(cid == cv)
                def _():
                    j0 = jp * 2
                    load_idx(j0, bufs[0])
                    load_idx(j0 + 1, bufs[1])
                    ha = fire_gathers(hc, s0t, s1t, d0t, d1t, bufs[0])
                    hb = fire_gathers(hc, s0t, s1t, d0t, d1t, bufs[1])
                    for h in ha:
                        h.wait()
                    compute_scatter(bufs[0])
                    for h in hb:
                        h.wait()
                    compute_scatter(bufs[1])
            return carry

        lax.fori_loop(0, NCH // 2, pair, 0)

        if NCH % 2 == 1:
            for cv, hc, s0t, s1t, d0t, d1t in variants:

                @pl.when(cid == cv)
                def _():
                    load_idx(NCH - 1, bufs[0])
                    ha = fire_gathers(hc, s0t, s1t, d0t, d1t, bufs[0])
                    for h in ha:
                        h.wait()
                    compute_scatter(bufs[0])
        plsc.subcore_barrier()

        for jj in range(5):
            rr = r0 + jj * 128

            @pl.when(cid == 0)
            def _():
                pltpu.sync_copy(un_acc.at[pl.ds(rr, 128)],
                                uo0.at[pl.ds(rr, 128)])

            @pl.when(cid == 1)
            def _():
                pltpu.sync_copy(un_acc.at[pl.ds(rr, 128)],
                                uo1.at[pl.ds(rr, 128)])


def _sc1(src, dst, ea, hp, sa_cols, da_cols, c16):
    f32 = jnp.float32
    k = pl.kernel(
        _sc1_body,
        mesh=_MESH,
        compiler_params=pltpu.CompilerParams(use_tc_tiling_on_sc=False),
        out_type=[jax.ShapeDtypeStruct((NP, 80), f32)] * 4,
        scratch_types=[
            pltpu.VMEM((64, 80), f32),
            pltpu.VMEM((16,), f32),
        ] + [
            pltpu.VMEM((C1,), jnp.int32),
            pltpu.VMEM((C1,), jnp.int32),
            pltpu.VMEM((C1,), f32),
            pltpu.VMEM((C1,), f32),
            pltpu.VMEM((C1,), f32),
            pltpu.VMEM((C1,), f32),
            pltpu.VMEM((C1,), f32),
            pltpu.VMEM((C1, 64), f32),
        ] * 2 + [
            pltpu.VMEM((C1, 80), f32),
            pltpu.VMEM_SHARED((NP, 80), f32),
            pltpu.SemaphoreType.DMA,
            pltpu.SemaphoreType.DMA,
        ],
    )
    return k(src, dst, ea, *hp, *sa_cols, *da_cols, c16)


def _sc2_body(src, dst, ea, t2, sa2, da2, c2v,
              un2_a, un2_b,
              zb, cbuf,
              srcv_0, dstv_0, eav_0, asrv_0, adsv_0, t2rows_0,
              srcv_1, dstv_1, eav_1, asrv_1, adsv_1, t2rows_1,
              wh2, un_acc, sem0, sem1):
    f32 = jnp.float32
    z16 = jnp.zeros((16,), f32)
    cid = lax.axis_index("c")
    sid = lax.axis_index("s")
    wid = cid * 16 + sid
    iota16 = lax.iota(jnp.int32, 16)
    lane0 = iota16 == 0
    zi = jnp.zeros((16,), jnp.int32)

    pltpu.sync_copy(c2v, cbuf)
    ch = _vtake(cbuf[pl.ds(0, 16)], zi)

    def zrow(i, carry):
        for j in range(3):
            zb[i, pl.ds(j * 16, 16)] = z16
        return carry

    lax.fori_loop(0, 64, zrow, 0)

    r0 = sid * ROWS_PT
    for jj in range(10):
        pltpu.sync_copy(zb, un_acc.at[pl.ds(r0 + jj * 64, 64)])
    plsc.subcore_barrier()

    ebase = wid * EPW2
    bufs = ((srcv_0, dstv_0, eav_0, asrv_0, adsv_0, t2rows_0, sem0),
            (srcv_1, dstv_1, eav_1, asrv_1, adsv_1, t2rows_1, sem1))
    NCH = EPW2 // C2

    def load_idx(jj, bu):
        base = ebase + jj * C2
        pltpu.sync_copy(src.at[pl.ds(base, C2)], bu[0])
        pltpu.sync_copy(dst.at[pl.ds(base, C2)], bu[1])
        pltpu.sync_copy(ea.at[pl.ds(base, C2)], bu[2])

    def fire_gathers(bu):
        sem = bu[6]
        return [pltpu.async_copy(t2.at[bu[0]], bu[5], sem),
                pltpu.async_copy(sa2.at[bu[0]], bu[3], sem),
                pltpu.async_copy(da2.at[bu[1]], bu[4], sem)]

    def compute_scatter(bu):
        eav_b, asrv_b, adsv_b, t2rows_b = bu[2], bu[3], bu[4], bu[5]

        def grp(g, carry2):
            sl16 = pl.ds(g * 16, 16)
            a = asrv_b[sl16] + adsv_b[sl16] + eav_b[sl16] * ch
            a = jnp.where(a >= 0.0, a, 0.2 * a)
            w = jnp.exp(a)
            for i in range(16):
                e = g * 16 + i
                wb = _vtake(w, _splat16(i))
                wh2[e, pl.ds(32, 16)] = jnp.where(lane0, wb, 0.0)
                wh2[e, pl.ds(0, 16)] = t2rows_b[e, pl.ds(0, 16)] * wb
                wh2[e, pl.ds(16, 16)] = t2rows_b[e, pl.ds(16, 16)] * wb
            return carry2

        lax.fori_loop(0, C2 // 16, grp, 0)
        pltpu.sync_copy(wh2, un_acc.at[bu[1]], add=True)

    def pair(jp, carry):
        j0 = jp * 2
        load_idx(j0, bufs[0])
        load_idx(j0 + 1, bufs[1])
        ha = fire_gathers(bufs[0])
        hb = fire_gathers(bufs[1])
        for h in ha:
            h.wait()
        compute_scatter(bufs[0])
        for h in hb:
            h.wait()
        compute_scatter(bufs[1])
        return carry

    lax.fori_loop(0, NCH // 2, pair, 0)
    load_idx(NCH - 1, bufs[0])
    for h in fire_gathers(bufs[0]):
        h.wait()
    compute_scatter(bufs[0])
    plsc.subcore_barrier()

    for jj in range(5):
        rr = r0 + jj * 128

        @pl.when(cid == 0)
        def _():
            pltpu.sync_copy(un_acc.at[pl.ds(rr, 128)],
                            un2_a.at[pl.ds(rr, 128)])

        @pl.when(cid == 1)
        def _():
            pltpu.sync_copy(un_acc.at[pl.ds(rr, 128)],
                            un2_b.at[pl.ds(rr, 128)])


def _sc2(src, dst, ea, t2, sa2, da2, c2v):
    f32 = jnp.float32
    k = pl.kernel(
        _sc2_body,
        mesh=_MESH,
        compiler_params=pltpu.CompilerParams(use_tc_tiling_on_sc=False),
        out_type=[jax.ShapeDtypeStruct((NP, 48), f32)] * 2,
        scratch_types=[
            pltpu.VMEM((64, 48), f32),
            pltpu.VMEM((16,), f32),
        ] + [
            pltpu.VMEM((C2,), jnp.int32),
            pltpu.VMEM((C2,), jnp.int32),
            pltpu.VMEM((C2,), f32),
            pltpu.VMEM((C2,), f32),
            pltpu.VMEM((C2,), f32),
            pltpu.VMEM((C2, 32), f32),
        ] * 2 + [
            pltpu.VMEM((C2, 48), f32),
            pltpu.VMEM_SHARED((NP, 48), f32),
            pltpu.SemaphoreType.DMA,
            pltpu.SemaphoreType.DMA,
        ],
    )
    return k(src, dst, ea, t2, sa2, da2, c2v)


def kernel(x, edge_index, edge_attr, batch, W1, a_src1, a_dst1, We1, a_edge1,
           b1, g1, be1, W2, a_src2, a_dst2, We2, a_edge2, b2, g2, be2, Wf, bf):
    f32 = jnp.float32
    src = edge_index[0]
    dst = edge_index[1]
    ea = edge_attr[:, 0]

    W1r = W1.reshape(D_IN, HEADS, HID)
    vsrc = jnp.einsum("dhj,hj->dh", W1r, a_src1)
    vdst = jnp.einsum("dhj,hj->dh", W1r, a_dst1)
    c1 = (We1.reshape(HEADS, HID) * a_edge1).sum(-1)
    wpieces = [W1r[:, 2 * p:2 * p + 2].reshape(D_IN, 64) for p in range(4)]
    c16 = jnp.zeros((16,), f32).at[0:8].set(c1)

    hp0, hp1, hp2, hp3, sa, da = _k1(x, wpieces, vsrc, vdst)
    sa_cols = [sa[:, h] for h in range(HEADS)]
    da_cols = [da[:, h] for h in range(HEADS)]
    uns = _sc1(src, dst, ea, (hp0, hp1, hp2, hp3), sa_cols, da_cols, c16)

    bs = [b1[64 * p:64 * p + 64].reshape(1, 64) for p in range(4)]
    s1, s2 = _k3a(uns, bs)
    npad = float(NP - N)
    mu = (s1[0] - npad * b1) / N
    var = (s2[0] - npad * b1 * b1) / N - mu * mu
    scale = g1 / jnp.sqrt(var + 1e-5)
    shift = be1 - mu * scale
    scs = [scale[64 * p:64 * p + 64].reshape(1, 64) for p in range(4)]
    shs = [shift[64 * p:64 * p + 64].reshape(1, 64) for p in range(4)]
    w2s = [W2[64 * p:64 * p + 64] for p in range(4)]

    a2 = jnp.zeros((HID, 8), f32).at[:, 0].set(a_src2[0])
    ad2 = jnp.zeros((HID, 8), f32).at[:, 0].set(a_dst2[0])
    c2 = (We2[0] * a_edge2[0]).sum()
    c2v = jnp.zeros((16,), f32).at[0].set(c2)

    t2, sa2, da2 = _k3b(uns, bs, scs, shs, w2s, a2, ad2)
    un2_a, un2_b = _sc2(src, dst, ea, t2, sa2[:, 0], da2[:, 0], c2v)

    b2r = b2.reshape(1, HID)
    batch_p = jnp.concatenate(
        [batch, jnp.full((NP - N,), G, jnp.int32)]).reshape(NP, 1)
    t1, t2s = _k5a(un2_a, un2_b, b2r)
    mu2 = (t1[0] - npad * b2) / N
    var2 = (t2s[0] - npad * b2 * b2) / N - mu2 * mu2
    scale2 = (g2 / jnp.sqrt(var2 + 1e-5)).reshape(1, HID)
    shift2 = (be2 - mu2 * (g2 / jnp.sqrt(var2 + 1e-5))).reshape(1, HID)

    _, _, out = _k5b(un2_a, un2_b, b2r, scale2, shift2,
                     batch_p, Wf, bf.reshape(1, 32))
    return out


# speedup vs baseline: 26.7355x; 1.0337x over previous
import functools

import jax
import jax.numpy as jnp
from jax import lax
from jax.experimental import pallas as pl
from jax.experimental.pallas import tpu as pltpu
from jax.experimental.pallas import tpu_sc as plsc

N = 10000
E = 320000
D_IN = 128
HID = 32
HEADS = 8
G = 64

NP = 10240
NB = 10
BLK = N // NB
BLKP = NP // NB


def _k1_body(x_ref, w0_ref, w1_ref, w2_ref, w3_ref, vs_ref, vd_ref,
             t0_ref, t1_ref, t2_ref, t3_ref, sa_ref, da_ref):
    xb = x_ref[...]
    f32 = jnp.float32
    for w_ref, t_ref in ((w0_ref, t0_ref), (w1_ref, t1_ref),
                         (w2_ref, t2_ref), (w3_ref, t3_ref)):
        t_ref[...] = jnp.dot(xb, w_ref[...], preferred_element_type=f32)
    sa_ref[...] = jnp.dot(xb, vs_ref[...], preferred_element_type=f32)
    da_ref[...] = jnp.dot(xb, vd_ref[...], preferred_element_type=f32)


def _k1(x, wpieces, vs, vd):
    f32 = jnp.float32
    wspec = pl.BlockSpec((128, 64), lambda i: (0, 0))
    vspec = pl.BlockSpec((128, 8), lambda i: (0, 0))
    tspec = pl.BlockSpec((BLK, 64), lambda i: (i, 0))
    aspec = pl.BlockSpec((BLK, 8), lambda i: (i, 0))
    tshape = jax.ShapeDtypeStruct((N, 64), f32)
    ashape = jax.ShapeDtypeStruct((N, 8), f32)
    return pl.pallas_call(
        _k1_body,
        grid=(NB,),
        in_specs=[pl.BlockSpec((BLK, 128), lambda i: (i, 0)),
                  wspec, wspec, wspec, wspec, vspec, vspec],
        out_specs=[tspec, tspec, tspec, tspec, aspec, aspec],
        out_shape=[tshape, tshape, tshape, tshape, ashape, ashape],
    )(x, *wpieces, vs, vd)


def _bn_stats_body(u0, u1, u2, u3, b0, b1r, b2r_, b3, s1_ref, s2_ref):
    i = pl.program_id(0)

    @pl.when(i == 0)
    def _():
        s1_ref[...] = jnp.zeros_like(s1_ref)
        s2_ref[...] = jnp.zeros_like(s2_ref)

    for p, (un_ref, b_ref) in enumerate(
            ((u0, b0), (u1, b1r), (u2, b2r_), (u3, b3))):
        for k in range(2):
            sl = slice(k * 32, (k + 1) * 32)
            v = (un_ref[:, sl] / (un_ref[:, 64 + k:65 + k] + 1e-16)
                 + b_ref[0:1, sl])
            osl = slice(p * 64 + k * 32, p * 64 + (k + 1) * 32)
            s1_ref[0:1, osl] += jnp.sum(v, axis=0, keepdims=True)
            s2_ref[0:1, osl] += jnp.sum(v * v, axis=0, keepdims=True)


def _k3a(uns, bs):
    f32 = jnp.float32
    uspec = pl.BlockSpec((BLKP, 80), lambda i: (i, 0))
    bspec = pl.BlockSpec((1, 64), lambda i: (0, 0))
    return pl.pallas_call(
        _bn_stats_body,
        grid=(NB,),
        in_specs=[uspec] * 4 + [bspec] * 4,
        out_specs=[
            pl.BlockSpec((8, 256), lambda i: (0, 0)),
            pl.BlockSpec((8, 256), lambda i: (0, 0)),
        ],
        out_shape=[
            jax.ShapeDtypeStruct((8, 256), f32),
            jax.ShapeDtypeStruct((8, 256), f32),
        ],
    )(*uns, *bs)


def _k3b_body(u0, u1, u2, u3, b0, b1r, b2r_, b3,
              sc0, sc1, sc2, sc3, sh0, sh1, sh2, sh3, w0, w1, w2_, w3,
              a2_ref, ad2_ref, t2_ref, sa2_ref, da2_ref):
    f32 = jnp.float32
    h2 = jnp.zeros((BLKP, HID), f32)
    for un_ref, b_ref, sc_ref, sh_ref, w2_ref in (
            (u0, b0, sc0, sh0, w0), (u1, b1r, sc1, sh1, w1),
            (u2, b2r_, sc2, sh2, w2_), (u3, b3, sc3, sh3, w3)):
        for k in range(2):
            sl = slice(k * 32, (k + 1) * 32)
            v = (un_ref[:, sl] / (un_ref[:, 64 + k:65 + k] + 1e-16)
                 + b_ref[0:1, sl])
            v = v * sc_ref[0:1, sl] + sh_ref[0:1, sl]
            v = jnp.maximum(v, 0.0)
            h2 = h2 + jnp.dot(v, w2_ref[sl, :], preferred_element_type=f32)
    t2_ref[...] = h2
    sa2_ref[...] = jnp.dot(h2, a2_ref[...], preferred_element_type=f32)
    da2_ref[...] = jnp.dot(h2, ad2_ref[...], preferred_element_type=f32)


def _k3b(uns, bs, scs, shs, w2s, a2, ad2):
    f32 = jnp.float32
    uspec = pl.BlockSpec((BLKP, 80), lambda i: (i, 0))
    bspec = pl.BlockSpec((1, 64), lambda i: (0, 0))
    wspec = pl.BlockSpec((64, 32), lambda i: (0, 0))
    return pl.pallas_call(
        _k3b_body,
        grid=(NB,),
        in_specs=[uspec] * 4 + [bspec] * 12 + [wspec] * 4 + [
            pl.BlockSpec((32, 8), lambda i: (0, 0)),
            pl.BlockSpec((32, 8), lambda i: (0, 0)),
        ],
        out_specs=[
            pl.BlockSpec((BLKP, 32), lambda i: (i, 0)),
            pl.BlockSpec((BLKP, 8), lambda i: (i, 0)),
            pl.BlockSpec((BLKP, 8), lambda i: (i, 0)),
        ],
        out_shape=[
            jax.ShapeDtypeStruct((NP, 32), f32),
            jax.ShapeDtypeStruct((NP, 8), f32),
            jax.ShapeDtypeStruct((NP, 8), f32),
        ],
    )(*uns, *bs, *scs, *shs, *w2s, a2, ad2)


def _bn2_stats_body(una_ref, unb_ref, b2_ref, s1_ref, s2_ref):
    i = pl.program_id(0)

    @pl.when(i == 0)
    def _():
        s1_ref[...] = jnp.zeros_like(s1_ref)
        s2_ref[...] = jnp.zeros_like(s2_ref)

    de = una_ref[:, 32:33] + unb_ref[:, 32:33] + 1e-16
    v = (una_ref[:, 0:32] + unb_ref[:, 0:32]) / de + b2_ref[0:1, :]
    s1_ref[0:1, :] += jnp.sum(v, axis=0, keepdims=True)
    s2_ref[0:1, :] += jnp.sum(v * v, axis=0, keepdims=True)


def _k5a(un2_a, un2_b, b2r):
    f32 = jnp.float32
    return pl.pallas_call(
        _bn2_stats_body,
        grid=(NB,),
        in_specs=[
            pl.BlockSpec((BLKP, 48), lambda i: (i, 0)),
            pl.BlockSpec((BLKP, 48), lambda i: (i, 0)),
            pl.BlockSpec((1, 32), lambda i: (0, 0)),
        ],
        out_specs=[
            pl.BlockSpec((8, 32), lambda i: (0, 0)),
            pl.BlockSpec((8, 32), lambda i: (0, 0)),
        ],
        out_shape=[
            jax.ShapeDtypeStruct((8, 32), f32),
            jax.ShapeDtypeStruct((8, 32), f32),
        ],
    )(un2_a, un2_b, b2r)


def _pool_body(una_ref, unb_ref, b2_ref, sc2_ref, sh2_ref,
               batch_ref, wf_ref, bf_ref, p_ref, cnt_ref, out_ref):
    i = pl.program_id(0)
    f32 = jnp.float32

    @pl.when(i == 0)
    def _():
        p_ref[...] = jnp.zeros_like(p_ref)
        cnt_ref[...] = jnp.zeros_like(cnt_ref)

    de = una_ref[:, 32:33] + unb_ref[:, 32:33] + 1e-16
    v = (una_ref[:, 0:32] + unb_ref[:, 0:32]) / de + b2_ref[0:1, :]
    h = jnp.maximum(v * sc2_ref[0:1, :] + sh2_ref[0:1, :], 0.0)
    oh = (batch_ref[...] == lax.broadcasted_iota(jnp.int32, (BLKP, G), 1)
          ).astype(f32)
    dn = (((0,), (0,)), ((), ()))
    p_ref[...] += lax.dot_general(oh, h, dn, preferred_element_type=f32)
    cnt_ref[...] += lax.dot_general(oh, jnp.ones((BLKP, 8), f32), dn,
                                    preferred_element_type=f32)

    @pl.when(i == NB - 1)
    def _():
        pooled = p_ref[...] / jnp.maximum(cnt_ref[:, 0:1], 1.0)
        out_ref[...] = (jnp.dot(pooled, wf_ref[...],
                                preferred_element_type=f32) + bf_ref[0:1, :])


def _k5b(un2_a, un2_b, b2r, sc2, sh2, batch2d, wf, bfr):
    f32 = jnp.float32
    row32 = pl.BlockSpec((1, 32), lambda i: (0, 0))
    return pl.pallas_call(
        _pool_body,
        grid=(NB,),
        in_specs=[
            pl.BlockSpec((BLKP, 48), lambda i: (i, 0)),
            pl.BlockSpec((BLKP, 48), lambda i: (i, 0)),
            row32, row32, row32,
            pl.BlockSpec((BLKP, 1), lambda i: (i, 0)),
            pl.BlockSpec((32, 32), lambda i: (0, 0)),
            row32,
        ],
        out_specs=[
            pl.BlockSpec((G, 32), lambda i: (0, 0)),
            pl.BlockSpec((G, 8), lambda i: (0, 0)),
            pl.BlockSpec((G, 32), lambda i: (0, 0)),
        ],
        out_shape=[
            jax.ShapeDtypeStruct((G, 32), f32),
            jax.ShapeDtypeStruct((G, 8), f32),
            jax.ShapeDtypeStruct((G, 32), f32),
        ],
    )(un2_a, un2_b, b2r, sc2, sh2, batch2d, wf, bfr)


_MESH = plsc.VectorSubcoreMesh(core_axis_name="c", subcore_axis_name="s")


def _vtake(x, idx):
    return jnp.take_along_axis(x, idx, axis=0, mode="promise_in_bounds")


def _splat16(i):
    return jnp.full((16,), i, jnp.int32)

EPT1 = E // 16
C1 = 160
ROWS_PT = NP // 16
EPW2 = E // 32
C2 = 400


def _sc1_body(src, dst, ea, h0, h1, h2, h3,
              sa0, sa1, sa2, sa3, sa4, sa5, sa6, sa7,
              da0, da1, da2, da3, da4, da5, da6, da7, c16,
              un0, un1, un2, un3,
              zb, cbuf,
              srcv_0, dstv_0, eav_0, asr0_0, asr1_0, ads0_0, ads1_0, trows_0,
              srcv_1, dstv_1, eav_1, asr0_1, asr1_1, ads0_1, ads1_1, trows_1,
              wh0, wh1, un_acc, sem0, sem1, sem_s):
    f32 = jnp.float32
    z16 = jnp.zeros((16,), f32)
    cid = lax.axis_index("c")
    sid = lax.axis_index("s")
    iota16 = lax.iota(jnp.int32, 16)
    lane0 = iota16 == 0
    lane1 = iota16 == 1
    zi = jnp.zeros((16,), jnp.int32)

    pltpu.sync_copy(c16, cbuf)
    cv16 = cbuf[pl.ds(0, 16)]

    def zrow(i, carry):
        for j in range(5):
            zb[i, pl.ds(j * 16, 16)] = z16
        return carry

    lax.fori_loop(0, 64, zrow, 0)

    r0 = sid * ROWS_PT
    ebase = sid * EPT1
    sas = (sa0, sa1, sa2, sa3, sa4, sa5, sa6, sa7)
    das = (da0, da1, da2, da3, da4, da5, da6, da7)
    bufs = ((srcv_0, dstv_0, eav_0, asr0_0, asr1_0, ads0_0, ads1_0,
             trows_0, sem0),
            (srcv_1, dstv_1, eav_1, asr0_1, asr1_1, ads0_1, ads1_1,
             trows_1, sem1))
    NCH = EPT1 // C1

    for q, (hc0, hc1, uo0, uo1) in enumerate(
            ((h0, h2, un0, un2), (h1, h3, un1, un3))):
        hco = cid * 4 + q * 2
        ch0 = _vtake(cv16, zi + hco)
        ch1 = _vtake(cv16, zi + (hco + 1))

        def load_idx(jj, bu):
            base = ebase + jj * C1
            pltpu.sync_copy(src.at[pl.ds(base, C1)], bu[0])
            pltpu.sync_copy(dst.at[pl.ds(base, C1)], bu[1])
            pltpu.sync_copy(ea.at[pl.ds(base, C1)], bu[2])

        def fire_gathers(hc, s0t, s1t, d0t, d1t, bu):
            sem = bu[8]
            return [pltpu.async_copy(hc.at[bu[0]], bu[7], sem),
                    pltpu.async_copy(s0t.at[bu[0]], bu[3], sem),
                    pltpu.async_copy(s1t.at[bu[0]], bu[4], sem),
                    pltpu.async_copy(d0t.at[bu[1]], bu[5], sem),
                    pltpu.async_copy(d1t.at[bu[1]], bu[6], sem)]

        def compute(bu, wh):
            eav_b, asr0_b, asr1_b = bu[2], bu[3], bu[4]
            ads0_b, ads1_b, trows_b = bu[5], bu[6], bu[7]

            def grp(g, carry2):
                sl16 = pl.ds(g * 16, 16)
                ev = eav_b[sl16]
                a0 = asr0_b[sl16] + ads0_b[sl16] + ev * ch0
                a1 = asr1_b[sl16] + ads1_b[sl16] + ev * ch1
                a0 = jnp.where(a0 >= 0.0, a0, 0.2 * a0)
                a1 = jnp.where(a1 >= 0.0, a1, 0.2 * a1)
                w0 = jnp.exp(a0)
                w1 = jnp.exp(a1)
                for i in range(16):
                    e = g * 16 + i
                    si = _splat16(i)
                    wb0 = _vtake(w0, si)
                    wb1 = _vtake(w1, si)
                    wh[e, pl.ds(64, 16)] = jnp.where(
                        lane0, wb0, jnp.where(lane1, wb1, 0.0))
                    for r in range(4):
                        sl = pl.ds(r * 16, 16)
                        wh[e, sl] = trows_b[e, sl] * (wb0 if r < 2 else wb1)
                return carry2

            lax.fori_loop(0, C1 // 16, grp, 0)

        for jj in range(10):
            pltpu.sync_copy(zb, un_acc.at[pl.ds(r0 + jj * 64, 64)])
        plsc.subcore_barrier()

        variants = (
            (0, hc0, sas[2 * q], sas[2 * q + 1],
             das[2 * q], das[2 * q + 1]),
            (1, hc1, sas[4 + 2 * q], sas[4 + 2 * q + 1],
             das[4 + 2 * q], das[4 + 2 * q + 1]),
        )

        def pair(jp, carry):
            for cv, hc, s0t, s1t, d0t, d1t in variants:

                @pl.when(cid == cv)
                def _():
                    j0 = jp * 2
                    load_idx(j0, bufs[0])
                    load_idx(j0 + 1, bufs[1])
                    ha = fire_gathers(hc, s0t, s1t, d0t, d1t, bufs[0])
                    hb = fire_gathers(hc, s0t, s1t, d0t, d1t, bufs[1])
                    for h in ha:
                        h.wait()
                    compute(bufs[0], wh0)
                    s0 = pltpu.async_copy(wh0, un_acc.at[bufs[0][1]], sem_s,
                                          add=True)
                    for h in hb:
                        h.wait()
                    compute(bufs[1], wh1)
                    s1 = pltpu.async_copy(wh1, un_acc.at[bufs[1][1]], sem_s,
                                          add=True)
                    s0.wait()
                    s1.wait()
            return carry

        lax.fori_loop(0, NCH // 2, pair, 0)

        if NCH % 2 == 1:
            for cv, hc, s0t, s1t, d0t, d1t in variants:

                @pl.when(cid == cv)
                def _():
                    load_idx(NCH - 1, bufs[0])
                    ha = fire_gathers(hc, s0t, s1t, d0t, d1t, bufs[0])
                    for h in ha:
                        h.wait()
                    compute(bufs[0], wh0)
                    pltpu.sync_copy(wh0, un_acc.at[bufs[0][1]], add=True)
        plsc.subcore_barrier()

        for jj in range(5):
            rr = r0 + jj * 128

            @pl.when(cid == 0)
            def _():
                pltpu.sync_copy(un_acc.at[pl.ds(rr, 128)],
                                uo0.at[pl.ds(rr, 128)])

            @pl.when(cid == 1)
            def _():
                pltpu.sync_copy(un_acc.at[pl.ds(rr, 128)],
                                uo1.at[pl.ds(rr, 128)])


def _sc1(src, dst, ea, hp, sa_cols, da_cols, c16):
    f32 = jnp.float32
    k = pl.kernel(
        _sc1_body,
        mesh=_MESH,
        compiler_params=pltpu.CompilerParams(use_tc_tiling_on_sc=False),
        out_type=[jax.ShapeDtypeStruct((NP, 80), f32)] * 4,
        scratch_types=[
            pltpu.VMEM((64, 80), f32),
            pltpu.VMEM((16,), f32),
        ] + [
            pltpu.VMEM((C1,), jnp.int32),
            pltpu.VMEM((C1,), jnp.int32),
            pltpu.VMEM((C1,), f32),
            pltpu.VMEM((C1,), f32),
            pltpu.VMEM((C1,), f32),
            pltpu.VMEM((C1,), f32),
            pltpu.VMEM((C1,), f32),
            pltpu.VMEM((C1, 64), f32),
        ] * 2 + [
            pltpu.VMEM((C1, 80), f32),
            pltpu.VMEM((C1, 80), f32),
            pltpu.VMEM_SHARED((NP, 80), f32),
            pltpu.SemaphoreType.DMA,
            pltpu.SemaphoreType.DMA,
            pltpu.SemaphoreType.DMA,
        ],
    )
    return k(src, dst, ea, *hp, *sa_cols, *da_cols, c16)


def _sc2_body(src, dst, ea, t2, sa2, da2, c2v,
              un2_a, un2_b,
              zb, cbuf,
              srcv_0, dstv_0, eav_0, asrv_0, adsv_0, t2rows_0,
              srcv_1, dstv_1, eav_1, asrv_1, adsv_1, t2rows_1,
              wh2, un_acc, sem0, sem1):
    f32 = jnp.float32
    z16 = jnp.zeros((16,), f32)
    cid = lax.axis_index("c")
    sid = lax.axis_index("s")
    wid = cid * 16 + sid
    iota16 = lax.iota(jnp.int32, 16)
    lane0 = iota16 == 0
    zi = jnp.zeros((16,), jnp.int32)

    pltpu.sync_copy(c2v, cbuf)
    ch = _vtake(cbuf[pl.ds(0, 16)], zi)

    def zrow(i, carry):
        for j in range(3):
            zb[i, pl.ds(j * 16, 16)] = z16
        return carry

    lax.fori_loop(0, 64, zrow, 0)

    r0 = sid * ROWS_PT
    for jj in range(10):
        pltpu.sync_copy(zb, un_acc.at[pl.ds(r0 + jj * 64, 64)])
    plsc.subcore_barrier()

    ebase = wid * EPW2
    bufs = ((srcv_0, dstv_0, eav_0, asrv_0, adsv_0, t2rows_0, sem0),
            (srcv_1, dstv_1, eav_1, asrv_1, adsv_1, t2rows_1, sem1))
    NCH = EPW2 // C2

    def load_idx(jj, bu):
        base = ebase + jj * C2
        pltpu.sync_copy(src.at[pl.ds(base, C2)], bu[0])
        pltpu.sync_copy(dst.at[pl.ds(base, C2)], bu[1])
        pltpu.sync_copy(ea.at[pl.ds(base, C2)], bu[2])

    def fire_gathers(bu):
        sem = bu[6]
        return [pltpu.async_copy(t2.at[bu[0]], bu[5], sem),
                pltpu.async_copy(sa2.at[bu[0]], bu[3], sem),
                pltpu.async_copy(da2.at[bu[1]], bu[4], sem)]

    def compute_scatter(bu):
        eav_b, asrv_b, adsv_b, t2rows_b = bu[2], bu[3], bu[4], bu[5]

        def grp(g, carry2):
            sl16 = pl.ds(g * 16, 16)
            a = asrv_b[sl16] + adsv_b[sl16] + eav_b[sl16] * ch
            a = jnp.where(a >= 0.0, a, 0.2 * a)
            w = jnp.exp(a)
            for i in range(16):
                e = g * 16 + i
                wb = _vtake(w, _splat16(i))
                wh2[e, pl.ds(32, 16)] = jnp.where(lane0, wb, 0.0)
                wh2[e, pl.ds(0, 16)] = t2rows_b[e, pl.ds(0, 16)] * wb
                wh2[e, pl.ds(16, 16)] = t2rows_b[e, pl.ds(16, 16)] * wb
            return carry2

        lax.fori_loop(0, C2 // 16, grp, 0)
        pltpu.sync_copy(wh2, un_acc.at[bu[1]], add=True)

    def pair(jp, carry):
        j0 = jp * 2
        load_idx(j0, bufs[0])
        load_idx(j0 + 1, bufs[1])
        ha = fire_gathers(bufs[0])
        hb = fire_gathers(bufs[1])
        for h in ha:
            h.wait()
        compute_scatter(bufs[0])
        for h in hb:
            h.wait()
        compute_scatter(bufs[1])
        return carry

    lax.fori_loop(0, NCH // 2, pair, 0)
    load_idx(NCH - 1, bufs[0])
    for h in fire_gathers(bufs[0]):
        h.wait()
    compute_scatter(bufs[0])
    plsc.subcore_barrier()

    for jj in range(5):
        rr = r0 + jj * 128

        @pl.when(cid == 0)
        def _():
            pltpu.sync_copy(un_acc.at[pl.ds(rr, 128)],
                            un2_a.at[pl.ds(rr, 128)])

        @pl.when(cid == 1)
        def _():
            pltpu.sync_copy(un_acc.at[pl.ds(rr, 128)],
                            un2_b.at[pl.ds(rr, 128)])


def _sc2(src, dst, ea, t2, sa2, da2, c2v):
    f32 = jnp.float32
    k = pl.kernel(
        _sc2_body,
        mesh=_MESH,
        compiler_params=pltpu.CompilerParams(use_tc_tiling_on_sc=False),
        out_type=[jax.ShapeDtypeStruct((NP, 48), f32)] * 2,
        scratch_types=[
            pltpu.VMEM((64, 48), f32),
            pltpu.VMEM((16,), f32),
        ] + [
            pltpu.VMEM((C2,), jnp.int32),
            pltpu.VMEM((C2,), jnp.int32),
            pltpu.VMEM((C2,), f32),
            pltpu.VMEM((C2,), f32),
            pltpu.VMEM((C2,), f32),
            pltpu.VMEM((C2, 32), f32),
        ] * 2 + [
            pltpu.VMEM((C2, 48), f32),
            pltpu.VMEM_SHARED((NP, 48), f32),
            pltpu.SemaphoreType.DMA,
            pltpu.SemaphoreType.DMA,
        ],
    )
    return k(src, dst, ea, t2, sa2, da2, c2v)


def kernel(x, edge_index, edge_attr, batch, W1, a_src1, a_dst1, We1, a_edge1,
           b1, g1, be1, W2, a_src2, a_dst2, We2, a_edge2, b2, g2, be2, Wf, bf):
    f32 = jnp.float32
    src = edge_index[0]
    dst = edge_index[1]
    ea = edge_attr[:, 0]

    W1r = W1.reshape(D_IN, HEADS, HID)
    vsrc = jnp.einsum("dhj,hj->dh", W1r, a_src1)
    vdst = jnp.einsum("dhj,hj->dh", W1r, a_dst1)
    c1 = (We1.reshape(HEADS, HID) * a_edge1).sum(-1)
    wpieces = [W1r[:, 2 * p:2 * p + 2].reshape(D_IN, 64) for p in range(4)]
    c16 = jnp.zeros((16,), f32).at[0:8].set(c1)

    hp0, hp1, hp2, hp3, sa, da = _k1(x, wpieces, vsrc, vdst)
    sa_cols = [sa[:, h] for h in range(HEADS)]
    da_cols = [da[:, h] for h in range(HEADS)]
    uns = _sc1(src, dst, ea, (hp0, hp1, hp2, hp3), sa_cols, da_cols, c16)

    bs = [b1[64 * p:64 * p + 64].reshape(1, 64) for p in range(4)]
    s1, s2 = _k3a(uns, bs)
    npad = float(NP - N)
    mu = (s1[0] - npad * b1) / N
    var = (s2[0] - npad * b1 * b1) / N - mu * mu
    scale = g1 / jnp.sqrt(var + 1e-5)
    shift = be1 - mu * scale
    scs = [scale[64 * p:64 * p + 64].reshape(1, 64) for p in range(4)]
    shs = [shift[64 * p:64 * p + 64].reshape(1, 64) for p in range(4)]
    w2s = [W2[64 * p:64 * p + 64] for p in range(4)]

    a2 = jnp.zeros((HID, 8), f32).at[:, 0].set(a_src2[0])
    ad2 = jnp.zeros((HID, 8), f32).at[:, 0].set(a_dst2[0])
    c2 = (We2[0] * a_edge2[0]).sum()
    c2v = jnp.zeros((16,), f32).at[0].set(c2)

    t2, sa2, da2 = _k3b(uns, bs, scs, shs, w2s, a2, ad2)
    un2_a, un2_b = _sc2(src, dst, ea, t2, sa2[:, 0], da2[:, 0], c2v)

    b2r = b2.reshape(1, HID)
    batch_p = jnp.concatenate(
        [batch, jnp.full((NP - N,), G, jnp.int32)]).reshape(NP, 1)
    t1, t2s = _k5a(un2_a, un2_b, b2r)
    mu2 = (t1[0] - npad * b2) / N
    var2 = (t2s[0] - npad * b2 * b2) / N - mu2 * mu2
    scale2 = (g2 / jnp.sqrt(var2 + 1e-5)).reshape(1, HID)
    shift2 = (be2 - mu2 * (g2 / jnp.sqrt(var2 + 1e-5))).reshape(1, HID)

    _, _, out = _k5b(un2_a, un2_b, b2r, scale2, shift2,
                     batch_p, Wf, bf.reshape(1, 32))
    return out


# speedup vs baseline: 26.8545x; 1.0045x over previous
import functools

import jax
import jax.numpy as jnp
from jax import lax
from jax.experimental import pallas as pl
from jax.experimental.pallas import tpu as pltpu
from jax.experimental.pallas import tpu_sc as plsc

N = 10000
E = 320000
D_IN = 128
HID = 32
HEADS = 8
G = 64

NP = 10240
NB = 10
BLK = N // NB
BLKP = NP // NB


def _k1_body(x_ref, w0_ref, w1_ref, w2_ref, w3_ref, vs_ref, vd_ref,
             t0_ref, t1_ref, t2_ref, t3_ref, sa_ref, da_ref):
    xb = x_ref[...]
    f32 = jnp.float32
    for w_ref, t_ref in ((w0_ref, t0_ref), (w1_ref, t1_ref),
                         (w2_ref, t2_ref), (w3_ref, t3_ref)):
        t_ref[...] = jnp.dot(xb, w_ref[...], preferred_element_type=f32)
    sa_ref[...] = jnp.dot(xb, vs_ref[...], preferred_element_type=f32)
    da_ref[...] = jnp.dot(xb, vd_ref[...], preferred_element_type=f32)


def _k1(x, wpieces, vs, vd):
    f32 = jnp.float32
    wspec = pl.BlockSpec((128, 64), lambda i: (0, 0))
    vspec = pl.BlockSpec((128, 8), lambda i: (0, 0))
    tspec = pl.BlockSpec((BLK, 64), lambda i: (i, 0))
    aspec = pl.BlockSpec((BLK, 8), lambda i: (i, 0))
    tshape = jax.ShapeDtypeStruct((N, 64), f32)
    ashape = jax.ShapeDtypeStruct((N, 8), f32)
    return pl.pallas_call(
        _k1_body,
        grid=(NB,),
        in_specs=[pl.BlockSpec((BLK, 128), lambda i: (i, 0)),
                  wspec, wspec, wspec, wspec, vspec, vspec],
        out_specs=[tspec, tspec, tspec, tspec, aspec, aspec],
        out_shape=[tshape, tshape, tshape, tshape, ashape, ashape],
    )(x, *wpieces, vs, vd)


def _bn_stats_body(u0, u1, u2, u3, b0, b1r, b2r_, b3, s1_ref, s2_ref):
    i = pl.program_id(0)

    @pl.when(i == 0)
    def _():
        s1_ref[...] = jnp.zeros_like(s1_ref)
        s2_ref[...] = jnp.zeros_like(s2_ref)

    for p, (un_ref, b_ref) in enumerate(
            ((u0, b0), (u1, b1r), (u2, b2r_), (u3, b3))):
        for k in range(2):
            sl = slice(k * 32, (k + 1) * 32)
            v = (un_ref[:, sl] / (un_ref[:, 64 + k:65 + k] + 1e-16)
                 + b_ref[0:1, sl])
            osl = slice(p * 64 + k * 32, p * 64 + (k + 1) * 32)
            s1_ref[0:1, osl] += jnp.sum(v, axis=0, keepdims=True)
            s2_ref[0:1, osl] += jnp.sum(v * v, axis=0, keepdims=True)


def _k3a(uns, bs):
    f32 = jnp.float32
    uspec = pl.BlockSpec((BLKP, 80), lambda i: (i, 0))
    bspec = pl.BlockSpec((1, 64), lambda i: (0, 0))
    return pl.pallas_call(
        _bn_stats_body,
        grid=(NB,),
        in_specs=[uspec] * 4 + [bspec] * 4,
        out_specs=[
            pl.BlockSpec((8, 256), lambda i: (0, 0)),
            pl.BlockSpec((8, 256), lambda i: (0, 0)),
        ],
        out_shape=[
            jax.ShapeDtypeStruct((8, 256), f32),
            jax.ShapeDtypeStruct((8, 256), f32),
        ],
    )(*uns, *bs)


def _k3b_body(u0, u1, u2, u3, b0, b1r, b2r_, b3,
              sc0, sc1, sc2, sc3, sh0, sh1, sh2, sh3, w0, w1, w2_, w3,
              a2_ref, ad2_ref, t2_ref, sa2_ref, da2_ref):
    f32 = jnp.float32
    h2 = jnp.zeros((BLKP, HID), f32)
    for un_ref, b_ref, sc_ref, sh_ref, w2_ref in (
            (u0, b0, sc0, sh0, w0), (u1, b1r, sc1, sh1, w1),
            (u2, b2r_, sc2, sh2, w2_), (u3, b3, sc3, sh3, w3)):
        for k in range(2):
            sl = slice(k * 32, (k + 1) * 32)
            v = (un_ref[:, sl] / (un_ref[:, 64 + k:65 + k] + 1e-16)
                 + b_ref[0:1, sl])
            v = v * sc_ref[0:1, sl] + sh_ref[0:1, sl]
            v = jnp.maximum(v, 0.0)
            h2 = h2 + jnp.dot(v, w2_ref[sl, :], preferred_element_type=f32)
    t2_ref[...] = h2
    sa2_ref[...] = jnp.dot(h2, a2_ref[...], preferred_element_type=f32)
    da2_ref[...] = jnp.dot(h2, ad2_ref[...], preferred_element_type=f32)


def _k3b(uns, bs, scs, shs, w2s, a2, ad2):
    f32 = jnp.float32
    uspec = pl.BlockSpec((BLKP, 80), lambda i: (i, 0))
    bspec = pl.BlockSpec((1, 64), lambda i: (0, 0))
    wspec = pl.BlockSpec((64, 32), lambda i: (0, 0))
    return pl.pallas_call(
        _k3b_body,
        grid=(NB,),
        in_specs=[uspec] * 4 + [bspec] * 12 + [wspec] * 4 + [
            pl.BlockSpec((32, 8), lambda i: (0, 0)),
            pl.BlockSpec((32, 8), lambda i: (0, 0)),
        ],
        out_specs=[
            pl.BlockSpec((BLKP, 32), lambda i: (i, 0)),
            pl.BlockSpec((BLKP, 8), lambda i: (i, 0)),
            pl.BlockSpec((BLKP, 8), lambda i: (i, 0)),
        ],
        out_shape=[
            jax.ShapeDtypeStruct((NP, 32), f32),
            jax.ShapeDtypeStruct((NP, 8), f32),
            jax.ShapeDtypeStruct((NP, 8), f32),
        ],
    )(*uns, *bs, *scs, *shs, *w2s, a2, ad2)


def _bn2_stats_body(una_ref, unb_ref, b2_ref, s1_ref, s2_ref):
    i = pl.program_id(0)

    @pl.when(i == 0)
    def _():
        s1_ref[...] = jnp.zeros_like(s1_ref)
        s2_ref[...] = jnp.zeros_like(s2_ref)

    de = una_ref[:, 32:33] + unb_ref[:, 32:33] + 1e-16
    v = (una_ref[:, 0:32] + unb_ref[:, 0:32]) / de + b2_ref[0:1, :]
    s1_ref[0:1, :] += jnp.sum(v, axis=0, keepdims=True)
    s2_ref[0:1, :] += jnp.sum(v * v, axis=0, keepdims=True)


def _k5a(un2_a, un2_b, b2r):
    f32 = jnp.float32
    return pl.pallas_call(
        _bn2_stats_body,
        grid=(NB,),
        in_specs=[
            pl.BlockSpec((BLKP, 48), lambda i: (i, 0)),
            pl.BlockSpec((BLKP, 48), lambda i: (i, 0)),
            pl.BlockSpec((1, 32), lambda i: (0, 0)),
        ],
        out_specs=[
            pl.BlockSpec((8, 32), lambda i: (0, 0)),
            pl.BlockSpec((8, 32), lambda i: (0, 0)),
        ],
        out_shape=[
            jax.ShapeDtypeStruct((8, 32), f32),
            jax.ShapeDtypeStruct((8, 32), f32),
        ],
    )(un2_a, un2_b, b2r)


def _pool_body(una_ref, unb_ref, b2_ref, sc2_ref, sh2_ref,
               batch_ref, wf_ref, bf_ref, p_ref, cnt_ref, out_ref):
    i = pl.program_id(0)
    f32 = jnp.float32

    @pl.when(i == 0)
    def _():
        p_ref[...] = jnp.zeros_like(p_ref)
        cnt_ref[...] = jnp.zeros_like(cnt_ref)

    de = una_ref[:, 32:33] + unb_ref[:, 32:33] + 1e-16
    v = (una_ref[:, 0:32] + unb_ref[:, 0:32]) / de + b2_ref[0:1, :]
    h = jnp.maximum(v * sc2_ref[0:1, :] + sh2_ref[0:1, :], 0.0)
    oh = (batch_ref[...] == lax.broadcasted_iota(jnp.int32, (BLKP, G), 1)
          ).astype(f32)
    dn = (((0,), (0,)), ((), ()))
    p_ref[...] += lax.dot_general(oh, h, dn, preferred_element_type=f32)
    cnt_ref[...] += lax.dot_general(oh, jnp.ones((BLKP, 8), f32), dn,
                                    preferred_element_type=f32)

    @pl.when(i == NB - 1)
    def _():
        pooled = p_ref[...] / jnp.maximum(cnt_ref[:, 0:1], 1.0)
        out_ref[...] = (jnp.dot(pooled, wf_ref[...],
                                preferred_element_type=f32) + bf_ref[0:1, :])


def _k5b(un2_a, un2_b, b2r, sc2, sh2, batch2d, wf, bfr):
    f32 = jnp.float32
    row32 = pl.BlockSpec((1, 32), lambda i: (0, 0))
    return pl.pallas_call(
        _pool_body,
        grid=(NB,),
        in_specs=[
            pl.BlockSpec((BLKP, 48), lambda i: (i, 0)),
            pl.BlockSpec((BLKP, 48), lambda i: (i, 0)),
            row32, row32, row32,
            pl.BlockSpec((BLKP, 1), lambda i: (i, 0)),
            pl.BlockSpec((32, 32), lambda i: (0, 0)),
            row32,
        ],
        out_specs=[
            pl.BlockSpec((G, 32), lambda i: (0, 0)),
            pl.BlockSpec((G, 8), lambda i: (0, 0)),
            pl.BlockSpec((G, 32), lambda i: (0, 0)),
        ],
        out_shape=[
            jax.ShapeDtypeStruct((G, 32), f32),
            jax.ShapeDtypeStruct((G, 8), f32),
            jax.ShapeDtypeStruct((G, 32), f32),
        ],
    )(un2_a, un2_b, b2r, sc2, sh2, batch2d, wf, bfr)


_MESH = plsc.VectorSubcoreMesh(core_axis_name="c", subcore_axis_name="s")


def _vtake(x, idx):
    return jnp.take_along_axis(x, idx, axis=0, mode="promise_in_bounds")


def _splat16(i):
    return jnp.full((16,), i, jnp.int32)

EPT1 = E // 16
C1 = 160
ROWS_PT = NP // 16
EPW2 = E // 32
C2 = 400


def _sc1_body(src, dst, ea, h0, h1, h2, h3,
              sa0, sa1, sa2, sa3, sa4, sa5, sa6, sa7,
              da0, da1, da2, da3, da4, da5, da6, da7, c16,
              un0, un1, un2, un3,
              zb, cbuf,
              srcv_0, dstv_0, eav_0, asr0_0, asr1_0, ads0_0, ads1_0, trows_0,
              srcv_1, dstv_1, eav_1, asr0_1, asr1_1, ads0_1, ads1_1, trows_1,
              wh0, wh1, un_acc, sem0, sem1, sem_s):
    f32 = jnp.float32
    z16 = jnp.zeros((16,), f32)
    cid = lax.axis_index("c")
    sid = lax.axis_index("s")
    iota16 = lax.iota(jnp.int32, 16)
    lane0 = iota16 == 0
    lane1 = iota16 == 1
    zi = jnp.zeros((16,), jnp.int32)

    pltpu.sync_copy(c16, cbuf)
    cv16 = cbuf[pl.ds(0, 16)]

    def zrow(i, carry):
        for j in range(5):
            zb[i, pl.ds(j * 16, 16)] = z16
        return carry

    lax.fori_loop(0, 64, zrow, 0)

    r0 = sid * ROWS_PT
    ebase = sid * EPT1
    sas = (sa0, sa1, sa2, sa3, sa4, sa5, sa6, sa7)
    das = (da0, da1, da2, da3, da4, da5, da6, da7)
    bufs = ((srcv_0, dstv_0, eav_0, asr0_0, asr1_0, ads0_0, ads1_0,
             trows_0, sem0),
            (srcv_1, dstv_1, eav_1, asr0_1, asr1_1, ads0_1, ads1_1,
             trows_1, sem1))
    NCH = EPT1 // C1

    for q, (hc0, hc1, uo0, uo1) in enumerate(
            ((h0, h2, un0, un2), (h1, h3, un1, un3))):
        hco = cid * 4 + q * 2
        ch0 = _vtake(cv16, zi + hco)
        ch1 = _vtake(cv16, zi + (hco + 1))

        def load_idx(jj, bu):
            base = ebase + jj * C1
            pltpu.sync_copy(src.at[pl.ds(base, C1)], bu[0])
            pltpu.sync_copy(dst.at[pl.ds(base, C1)], bu[1])
            pltpu.sync_copy(ea.at[pl.ds(base, C1)], bu[2])

        def fire_gathers(hc, s0t, s1t, d0t, d1t, bu):
            sem = bu[8]
            return [pltpu.async_copy(hc.at[bu[0]], bu[7], sem),
                    pltpu.async_copy(s0t.at[bu[0]], bu[3], sem),
                    pltpu.async_copy(s1t.at[bu[0]], bu[4], sem),
                    pltpu.async_copy(d0t.at[bu[1]], bu[5], sem),
                    pltpu.async_copy(d1t.at[bu[1]], bu[6], sem)]

        def compute(bu, wh):
            eav_b, asr0_b, asr1_b = bu[2], bu[3], bu[4]
            ads0_b, ads1_b, trows_b = bu[5], bu[6], bu[7]

            def grp(g, carry2):
                sl16 = pl.ds(g * 16, 16)
                ev = eav_b[sl16]
                a0 = asr0_b[sl16] + ads0_b[sl16] + ev * ch0
                a1 = asr1_b[sl16] + ads1_b[sl16] + ev * ch1
                a0 = jnp.where(a0 >= 0.0, a0, 0.2 * a0)
                a1 = jnp.where(a1 >= 0.0, a1, 0.2 * a1)
                w0 = jnp.exp(a0)
                w1 = jnp.exp(a1)
                for i in range(16):
                    e = g * 16 + i
                    si = _splat16(i)
                    wb0 = _vtake(w0, si)
                    wb1 = _vtake(w1, si)
                    wh[e, pl.ds(64, 16)] = jnp.where(
                        lane0, wb0, jnp.where(lane1, wb1, 0.0))
                    for r in range(4):
                        sl = pl.ds(r * 16, 16)
                        wh[e, sl] = trows_b[e, sl] * (wb0 if r < 2 else wb1)
                return carry2

            lax.fori_loop(0, C1 // 16, grp, 0)

        for jj in range(10):
            pltpu.sync_copy(zb, un_acc.at[pl.ds(r0 + jj * 64, 64)])
        plsc.subcore_barrier()

        variants = (
            (0, hc0, sas[2 * q], sas[2 * q + 1],
             das[2 * q], das[2 * q + 1]),
            (1, hc1, sas[4 + 2 * q], sas[4 + 2 * q + 1],
             das[4 + 2 * q], das[4 + 2 * q + 1]),
        )

        def pair(jp, carry):
            for cv, hc, s0t, s1t, d0t, d1t in variants:

                @pl.when(cid == cv)
                def _():
                    j0 = jp * 2
                    load_idx(j0, bufs[0])
                    load_idx(j0 + 1, bufs[1])
                    ha = fire_gathers(hc, s0t, s1t, d0t, d1t, bufs[0])
                    hb = fire_gathers(hc, s0t, s1t, d0t, d1t, bufs[1])
                    for h in ha:
                        h.wait()
                    compute(bufs[0], wh0)
                    s0 = pltpu.async_copy(wh0, un_acc.at[bufs[0][1]], sem_s,
                                          add=True)
                    for h in hb:
                        h.wait()
                    compute(bufs[1], wh1)
                    s1 = pltpu.async_copy(wh1, un_acc.at[bufs[1][1]], sem_s,
                                          add=True)
                    s0.wait()
                    s1.wait()
            return carry

        lax.fori_loop(0, NCH // 2, pair, 0)

        if NCH % 2 == 1:
            for cv, hc, s0t, s1t, d0t, d1t in variants:

                @pl.when(cid == cv)
                def _():
                    load_idx(NCH - 1, bufs[0])
                    ha = fire_gathers(hc, s0t, s1t, d0t, d1t, bufs[0])
                    for h in ha:
                        h.wait()
                    compute(bufs[0], wh0)
                    pltpu.sync_copy(wh0, un_acc.at[bufs[0][1]], add=True)
        plsc.subcore_barrier()

        for jj in range(5):
            rr = r0 + jj * 128

            @pl.when(cid == 0)
            def _():
                pltpu.sync_copy(un_acc.at[pl.ds(rr, 128)],
                                uo0.at[pl.ds(rr, 128)])

            @pl.when(cid == 1)
            def _():
                pltpu.sync_copy(un_acc.at[pl.ds(rr, 128)],
                                uo1.at[pl.ds(rr, 128)])


def _sc1(src, dst, ea, hp, sa_cols, da_cols, c16):
    f32 = jnp.float32
    k = pl.kernel(
        _sc1_body,
        mesh=_MESH,
        compiler_params=pltpu.CompilerParams(use_tc_tiling_on_sc=False),
        out_type=[jax.ShapeDtypeStruct((NP, 80), f32)] * 4,
        scratch_types=[
            pltpu.VMEM((64, 80), f32),
            pltpu.VMEM((16,), f32),
        ] + [
            pltpu.VMEM((C1,), jnp.int32),
            pltpu.VMEM((C1,), jnp.int32),
            pltpu.VMEM((C1,), f32),
            pltpu.VMEM((C1,), f32),
            pltpu.VMEM((C1,), f32),
            pltpu.VMEM((C1,), f32),
            pltpu.VMEM((C1,), f32),
            pltpu.VMEM((C1, 64), f32),
        ] * 2 + [
            pltpu.VMEM((C1, 80), f32),
            pltpu.VMEM((C1, 80), f32),
            pltpu.VMEM_SHARED((NP, 80), f32),
            pltpu.SemaphoreType.DMA,
            pltpu.SemaphoreType.DMA,
            pltpu.SemaphoreType.DMA,
        ],
    )
    return k(src, dst, ea, *hp, *sa_cols, *da_cols, c16)


def _sc2_body(src, dst, ea, t2, sa2, da2, c2v,
              un2_a, un2_b,
              zb, cbuf,
              srcv_0, dstv_0, eav_0, asrv_0, adsv_0, t2rows_0,
              srcv_1, dstv_1, eav_1, asrv_1, adsv_1, t2rows_1,
              wh20, wh21, un_acc, sem0, sem1, sem_s):
    f32 = jnp.float32
    z16 = jnp.zeros((16,), f32)
    cid = lax.axis_index("c")
    sid = lax.axis_index("s")
    wid = cid * 16 + sid
    iota16 = lax.iota(jnp.int32, 16)
    lane0 = iota16 == 0
    zi = jnp.zeros((16,), jnp.int32)

    pltpu.sync_copy(c2v, cbuf)
    ch = _vtake(cbuf[pl.ds(0, 16)], zi)

    def zrow(i, carry):
        for j in range(3):
            zb[i, pl.ds(j * 16, 16)] = z16
        return carry

    lax.fori_loop(0, 64, zrow, 0)

    r0 = sid * ROWS_PT
    for jj in range(10):
        pltpu.sync_copy(zb, un_acc.at[pl.ds(r0 + jj * 64, 64)])
    plsc.subcore_barrier()

    ebase = wid * EPW2
    bufs = ((srcv_0, dstv_0, eav_0, asrv_0, adsv_0, t2rows_0, sem0),
            (srcv_1, dstv_1, eav_1, asrv_1, adsv_1, t2rows_1, sem1))
    NCH = EPW2 // C2

    def load_idx(jj, bu):
        base = ebase + jj * C2
        pltpu.sync_copy(src.at[pl.ds(base, C2)], bu[0])
        pltpu.sync_copy(dst.at[pl.ds(base, C2)], bu[1])
        pltpu.sync_copy(ea.at[pl.ds(base, C2)], bu[2])

    def fire_gathers(bu):
        sem = bu[6]
        return [pltpu.async_copy(t2.at[bu[0]], bu[5], sem),
                pltpu.async_copy(sa2.at[bu[0]], bu[3], sem),
                pltpu.async_copy(da2.at[bu[1]], bu[4], sem)]

    def compute(bu, wh2):
        eav_b, asrv_b, adsv_b, t2rows_b = bu[2], bu[3], bu[4], bu[5]

        def grp(g, carry2):
            sl16 = pl.ds(g * 16, 16)
            a = asrv_b[sl16] + adsv_b[sl16] + eav_b[sl16] * ch
            a = jnp.where(a >= 0.0, a, 0.2 * a)
            w = jnp.exp(a)
            for i in range(16):
                e = g * 16 + i
                wb = _vtake(w, _splat16(i))
                wh2[e, pl.ds(32, 16)] = jnp.where(lane0, wb, 0.0)
                wh2[e, pl.ds(0, 16)] = t2rows_b[e, pl.ds(0, 16)] * wb
                wh2[e, pl.ds(16, 16)] = t2rows_b[e, pl.ds(16, 16)] * wb
            return carry2

        lax.fori_loop(0, C2 // 16, grp, 0)

    def pair(jp, carry):
        j0 = jp * 2
        load_idx(j0, bufs[0])
        load_idx(j0 + 1, bufs[1])
        ha = fire_gathers(bufs[0])
        hb = fire_gathers(bufs[1])
        for h in ha:
            h.wait()
        compute(bufs[0], wh20)
        s0 = pltpu.async_copy(wh20, un_acc.at[bufs[0][1]], sem_s, add=True)
        for h in hb:
            h.wait()
        compute(bufs[1], wh21)
        s1 = pltpu.async_copy(wh21, un_acc.at[bufs[1][1]], sem_s, add=True)
        s0.wait()
        s1.wait()
        return carry

    lax.fori_loop(0, NCH // 2, pair, 0)
    load_idx(NCH - 1, bufs[0])
    for h in fire_gathers(bufs[0]):
        h.wait()
    compute(bufs[0], wh20)
    pltpu.sync_copy(wh20, un_acc.at[bufs[0][1]], add=True)
    plsc.subcore_barrier()

    for jj in range(5):
        rr = r0 + jj * 128

        @pl.when(cid == 0)
        def _():
            pltpu.sync_copy(un_acc.at[pl.ds(rr, 128)],
                            un2_a.at[pl.ds(rr, 128)])

        @pl.when(cid == 1)
        def _():
            pltpu.sync_copy(un_acc.at[pl.ds(rr, 128)],
                            un2_b.at[pl.ds(rr, 128)])


def _sc2(src, dst, ea, t2, sa2, da2, c2v):
    f32 = jnp.float32
    k = pl.kernel(
        _sc2_body,
        mesh=_MESH,
        compiler_params=pltpu.CompilerParams(use_tc_tiling_on_sc=False),
        out_type=[jax.ShapeDtypeStruct((NP, 48), f32)] * 2,
        scratch_types=[
            pltpu.VMEM((64, 48), f32),
            pltpu.VMEM((16,), f32),
        ] + [
            pltpu.VMEM((C2,), jnp.int32),
            pltpu.VMEM((C2,), jnp.int32),
            pltpu.VMEM((C2,), f32),
            pltpu.VMEM((C2,), f32),
            pltpu.VMEM((C2,), f32),
            pltpu.VMEM((C2, 32), f32),
        ] * 2 + [
            pltpu.VMEM((C2, 48), f32),
            pltpu.VMEM((C2, 48), f32),
            pltpu.VMEM_SHARED((NP, 48), f32),
            pltpu.SemaphoreType.DMA,
            pltpu.SemaphoreType.DMA,
            pltpu.SemaphoreType.DMA,
        ],
    )
    return k(src, dst, ea, t2, sa2, da2, c2v)


def kernel(x, edge_index, edge_attr, batch, W1, a_src1, a_dst1, We1, a_edge1,
           b1, g1, be1, W2, a_src2, a_dst2, We2, a_edge2, b2, g2, be2, Wf, bf):
    f32 = jnp.float32
    src = edge_index[0]
    dst = edge_index[1]
    ea = edge_attr[:, 0]

    W1r = W1.reshape(D_IN, HEADS, HID)
    vsrc = jnp.einsum("dhj,hj->dh", W1r, a_src1)
    vdst = jnp.einsum("dhj,hj->dh", W1r, a_dst1)
    c1 = (We1.reshape(HEADS, HID) * a_edge1).sum(-1)
    wpieces = [W1r[:, 2 * p:2 * p + 2].reshape(D_IN, 64) for p in range(4)]
    c16 = jnp.zeros((16,), f32).at[0:8].set(c1)

    hp0, hp1, hp2, hp3, sa, da = _k1(x, wpieces, vsrc, vdst)
    sa_cols = [sa[:, h] for h in range(HEADS)]
    da_cols = [da[:, h] for h in range(HEADS)]
    uns = _sc1(src, dst, ea, (hp0, hp1, hp2, hp3), sa_cols, da_cols, c16)

    bs = [b1[64 * p:64 * p + 64].reshape(1, 64) for p in range(4)]
    s1, s2 = _k3a(uns, bs)
    npad = float(NP - N)
    mu = (s1[0] - npad * b1) / N
    var = (s2[0] - npad * b1 * b1) / N - mu * mu
    scale = g1 / jnp.sqrt(var + 1e-5)
    shift = be1 - mu * scale
    scs = [scale[64 * p:64 * p + 64].reshape(1, 64) for p in range(4)]
    shs = [shift[64 * p:64 * p + 64].reshape(1, 64) for p in range(4)]
    w2s = [W2[64 * p:64 * p + 64] for p in range(4)]

    a2 = jnp.zeros((HID, 8), f32).at[:, 0].set(a_src2[0])
    ad2 = jnp.zeros((HID, 8), f32).at[:, 0].set(a_dst2[0])
    c2 = (We2[0] * a_edge2[0]).sum()
    c2v = jnp.zeros((16,), f32).at[0].set(c2)

    t2, sa2, da2 = _k3b(uns, bs, scs, shs, w2s, a2, ad2)
    un2_a, un2_b = _sc2(src, dst, ea, t2, sa2[:, 0], da2[:, 0], c2v)

    b2r = b2.reshape(1, HID)
    batch_p = jnp.concatenate(
        [batch, jnp.full((NP - N,), G, jnp.int32)]).reshape(NP, 1)
    t1, t2s = _k5a(un2_a, un2_b, b2r)
    mu2 = (t1[0] - npad * b2) / N
    var2 = (t2s[0] - npad * b2 * b2) / N - mu2 * mu2
    scale2 = (g2 / jnp.sqrt(var2 + 1e-5)).reshape(1, HID)
    shift2 = (be2 - mu2 * (g2 / jnp.sqrt(var2 + 1e-5))).reshape(1, HID)

    _, _, out = _k5b(un2_a, un2_b, b2r, scale2, shift2,
                     batch_p, Wf, bf.reshape(1, 32))
    return out


# speedup vs baseline: 26.8654x; 1.0004x over previous
import jax
import jax.numpy as jnp
from jax import lax
from jax.experimental import pallas as pl
from jax.experimental.pallas import tpu as pltpu
from jax.experimental.pallas import tpu_sc as plsc

N = 10000
E = 320000
D_IN = 128
HID = 32
HEADS = 8
G = 64

NP = 10240
NB = 10
BLK = N // NB
BLKP = NP // NB


def _k1_body(x_ref, w0_ref, w1_ref, w2_ref, w3_ref, vs_ref, vd_ref,
             t0_ref, t1_ref, t2_ref, t3_ref, sa_ref, da_ref):
    xb = x_ref[...]
    f32 = jnp.float32
    for w_ref, t_ref in ((w0_ref, t0_ref), (w1_ref, t1_ref),
                         (w2_ref, t2_ref), (w3_ref, t3_ref)):
        t_ref[...] = jnp.dot(xb, w_ref[...], preferred_element_type=f32)
    sa_ref[...] = jnp.dot(xb, vs_ref[...], preferred_element_type=f32)
    da_ref[...] = jnp.dot(xb, vd_ref[...], preferred_element_type=f32)


def _k1(x, wpieces, vs, vd):
    f32 = jnp.float32
    wspec = pl.BlockSpec((128, 64), lambda i: (0, 0))
    vspec = pl.BlockSpec((128, 8), lambda i: (0, 0))
    tspec = pl.BlockSpec((BLK, 64), lambda i: (i, 0))
    aspec = pl.BlockSpec((BLK, 8), lambda i: (i, 0))
    tshape = jax.ShapeDtypeStruct((N, 64), f32)
    ashape = jax.ShapeDtypeStruct((N, 8), f32)
    return pl.pallas_call(
        _k1_body,
        grid=(NB,),
        in_specs=[pl.BlockSpec((BLK, 128), lambda i: (i, 0)),
                  wspec, wspec, wspec, wspec, vspec, vspec],
        out_specs=[tspec, tspec, tspec, tspec, aspec, aspec],
        out_shape=[tshape, tshape, tshape, tshape, ashape, ashape],
    )(x, *wpieces, vs, vd)


def _bn_stats_body(u0, u1, u2, u3, b0, b1r, b2r_, b3, s1_ref, s2_ref):
    i = pl.program_id(0)

    @pl.when(i == 0)
    def _():
        s1_ref[...] = jnp.zeros_like(s1_ref)
        s2_ref[...] = jnp.zeros_like(s2_ref)

    for p, (un_ref, b_ref) in enumerate(
            ((u0, b0), (u1, b1r), (u2, b2r_), (u3, b3))):
        for k in range(2):
            sl = slice(k * 32, (k + 1) * 32)
            v = (un_ref[:, sl] / (un_ref[:, 64 + k:65 + k] + 1e-16)
                 + b_ref[0:1, sl])
            osl = slice(p * 64 + k * 32, p * 64 + (k + 1) * 32)
            s1_ref[0:1, osl] += jnp.sum(v, axis=0, keepdims=True)
            s2_ref[0:1, osl] += jnp.sum(v * v, axis=0, keepdims=True)


def _k3a(uns, bs):
    f32 = jnp.float32
    uspec = pl.BlockSpec((BLKP, 80), lambda i: (i, 0))
    bspec = pl.BlockSpec((1, 64), lambda i: (0, 0))
    return pl.pallas_call(
        _bn_stats_body,
        grid=(NB,),
        in_specs=[uspec] * 4 + [bspec] * 4,
        out_specs=[
            pl.BlockSpec((8, 256), lambda i: (0, 0)),
            pl.BlockSpec((8, 256), lambda i: (0, 0)),
        ],
        out_shape=[
            jax.ShapeDtypeStruct((8, 256), f32),
            jax.ShapeDtypeStruct((8, 256), f32),
        ],
    )(*uns, *bs)


def _k3b_body(u0, u1, u2, u3, b0, b1r, b2r_, b3,
              sc0, sc1, sc2, sc3, sh0, sh1, sh2, sh3, w0, w1, w2_, w3,
              a2_ref, ad2_ref, t2_ref, sa2_ref, da2_ref):
    f32 = jnp.float32
    h2 = jnp.zeros((BLKP, HID), f32)
    for un_ref, b_ref, sc_ref, sh_ref, w2_ref in (
            (u0, b0, sc0, sh0, w0), (u1, b1r, sc1, sh1, w1),
            (u2, b2r_, sc2, sh2, w2_), (u3, b3, sc3, sh3, w3)):
        for k in range(2):
            sl = slice(k * 32, (k + 1) * 32)
            v = (un_ref[:, sl] / (un_ref[:, 64 + k:65 + k] + 1e-16)
                 + b_ref[0:1, sl])
            v = v * sc_ref[0:1, sl] + sh_ref[0:1, sl]
            v = jnp.maximum(v, 0.0)
            h2 = h2 + jnp.dot(v, w2_ref[sl, :], preferred_element_type=f32)
    t2_ref[...] = h2
    sa2_ref[...] = jnp.dot(h2, a2_ref[...], preferred_element_type=f32)
    da2_ref[...] = jnp.dot(h2, ad2_ref[...], preferred_element_type=f32)


def _k3b(uns, bs, scs, shs, w2s, a2, ad2):
    f32 = jnp.float32
    uspec = pl.BlockSpec((BLKP, 80), lambda i: (i, 0))
    bspec = pl.BlockSpec((1, 64), lambda i: (0, 0))
    wspec = pl.BlockSpec((64, 32), lambda i: (0, 0))
    return pl.pallas_call(
        _k3b_body,
        grid=(NB,),
        in_specs=[uspec] * 4 + [bspec] * 12 + [wspec] * 4 + [
            pl.BlockSpec((32, 8), lambda i: (0, 0)),
            pl.BlockSpec((32, 8), lambda i: (0, 0)),
        ],
        out_specs=[
            pl.BlockSpec((BLKP, 32), lambda i: (i, 0)),
            pl.BlockSpec((BLKP, 8), lambda i: (i, 0)),
            pl.BlockSpec((BLKP, 8), lambda i: (i, 0)),
        ],
        out_shape=[
            jax.ShapeDtypeStruct((NP, 32), f32),
            jax.ShapeDtypeStruct((NP, 8), f32),
            jax.ShapeDtypeStruct((NP, 8), f32),
        ],
    )(*uns, *bs, *scs, *shs, *w2s, a2, ad2)


def _bn2_stats_body(una_ref, unb_ref, b2_ref, s1_ref, s2_ref):
    i = pl.program_id(0)

    @pl.when(i == 0)
    def _():
        s1_ref[...] = jnp.zeros_like(s1_ref)
        s2_ref[...] = jnp.zeros_like(s2_ref)

    de = una_ref[:, 32:33] + unb_ref[:, 32:33] + 1e-16
    v = (una_ref[:, 0:32] + unb_ref[:, 0:32]) / de + b2_ref[0:1, :]
    s1_ref[0:1, :] += jnp.sum(v, axis=0, keepdims=True)
    s2_ref[0:1, :] += jnp.sum(v * v, axis=0, keepdims=True)


def _k5a(un2_a, un2_b, b2r):
    f32 = jnp.float32
    return pl.pallas_call(
        _bn2_stats_body,
        grid=(NB,),
        in_specs=[
            pl.BlockSpec((BLKP, 48), lambda i: (i, 0)),
            pl.BlockSpec((BLKP, 48), lambda i: (i, 0)),
            pl.BlockSpec((1, 32), lambda i: (0, 0)),
        ],
        out_specs=[
            pl.BlockSpec((8, 32), lambda i: (0, 0)),
            pl.BlockSpec((8, 32), lambda i: (0, 0)),
        ],
        out_shape=[
            jax.ShapeDtypeStruct((8, 32), f32),
            jax.ShapeDtypeStruct((8, 32), f32),
        ],
    )(un2_a, un2_b, b2r)


def _pool_body(una_ref, unb_ref, b2_ref, sc2_ref, sh2_ref,
               batch_ref, wf_ref, bf_ref, p_ref, cnt_ref, out_ref):
    i = pl.program_id(0)
    f32 = jnp.float32

    @pl.when(i == 0)
    def _():
        p_ref[...] = jnp.zeros_like(p_ref)
        cnt_ref[...] = jnp.zeros_like(cnt_ref)

    de = una_ref[:, 32:33] + unb_ref[:, 32:33] + 1e-16
    v = (una_ref[:, 0:32] + unb_ref[:, 0:32]) / de + b2_ref[0:1, :]
    h = jnp.maximum(v * sc2_ref[0:1, :] + sh2_ref[0:1, :], 0.0)
    oh = (batch_ref[...] == lax.broadcasted_iota(jnp.int32, (BLKP, G), 1)
          ).astype(f32)
    dn = (((0,), (0,)), ((), ()))
    p_ref[...] += lax.dot_general(oh, h, dn, preferred_element_type=f32)
    cnt_ref[...] += lax.dot_general(oh, jnp.ones((BLKP, 8), f32), dn,
                                    preferred_element_type=f32)

    @pl.when(i == NB - 1)
    def _():
        pooled = p_ref[...] / jnp.maximum(cnt_ref[:, 0:1], 1.0)
        out_ref[...] = (jnp.dot(pooled, wf_ref[...],
                                preferred_element_type=f32) + bf_ref[0:1, :])


def _k5b(un2_a, un2_b, b2r, sc2, sh2, batch2d, wf, bfr):
    f32 = jnp.float32
    row32 = pl.BlockSpec((1, 32), lambda i: (0, 0))
    return pl.pallas_call(
        _pool_body,
        grid=(NB,),
        in_specs=[
            pl.BlockSpec((BLKP, 48), lambda i: (i, 0)),
            pl.BlockSpec((BLKP, 48), lambda i: (i, 0)),
            row32, row32, row32,
            pl.BlockSpec((BLKP, 1), lambda i: (i, 0)),
            pl.BlockSpec((32, 32), lambda i: (0, 0)),
            row32,
        ],
        out_specs=[
            pl.BlockSpec((G, 32), lambda i: (0, 0)),
            pl.BlockSpec((G, 8), lambda i: (0, 0)),
            pl.BlockSpec((G, 32), lambda i: (0, 0)),
        ],
        out_shape=[
            jax.ShapeDtypeStruct((G, 32), f32),
            jax.ShapeDtypeStruct((G, 8), f32),
            jax.ShapeDtypeStruct((G, 32), f32),
        ],
    )(un2_a, un2_b, b2r, sc2, sh2, batch2d, wf, bfr)


_MESH = plsc.VectorSubcoreMesh(core_axis_name="c", subcore_axis_name="s")


def _vtake(x, idx):
    return jnp.take_along_axis(x, idx, axis=0, mode="promise_in_bounds")


def _splat16(i):
    return jnp.full((16,), i, jnp.int32)

EPT1 = E // 16
C1 = 160
ROWS_PT = NP // 16
EPW2 = E // 32
C2 = 400


def _sc1_body(src, dst, ea, h0, h1, h2, h3,
              sa0, sa1, sa2, sa3, sa4, sa5, sa6, sa7,
              da0, da1, da2, da3, da4, da5, da6, da7, c16,
              un0, un1, un2, un3,
              zb, cbuf,
              srcv_0, dstv_0, eav_0, asr0_0, asr1_0, ads0_0, ads1_0, trows_0,
              srcv_1, dstv_1, eav_1, asr0_1, asr1_1, ads0_1, ads1_1, trows_1,
              wh0, wh1, un_acc, sem0, sem1, sem_s):
    f32 = jnp.float32
    z16 = jnp.zeros((16,), f32)
    cid = lax.axis_index("c")
    sid = lax.axis_index("s")
    iota16 = lax.iota(jnp.int32, 16)
    lane0 = iota16 == 0
    lane1 = iota16 == 1
    zi = jnp.zeros((16,), jnp.int32)

    pltpu.sync_copy(c16, cbuf)
    cv16 = cbuf[pl.ds(0, 16)]

    def zrow(i, carry):
        for j in range(5):
            zb[i, pl.ds(j * 16, 16)] = z16
        return carry

    lax.fori_loop(0, 64, zrow, 0)

    r0 = sid * ROWS_PT
    ebase = sid * EPT1
    sas = (sa0, sa1, sa2, sa3, sa4, sa5, sa6, sa7)
    das = (da0, da1, da2, da3, da4, da5, da6, da7)
    bufs = ((srcv_0, dstv_0, eav_0, asr0_0, asr1_0, ads0_0, ads1_0,
             trows_0, sem0),
            (srcv_1, dstv_1, eav_1, asr0_1, asr1_1, ads0_1, ads1_1,
             trows_1, sem1))
    NCH = EPT1 // C1

    for q, (hc0, hc1, uo0, uo1) in enumerate(
            ((h0, h2, un0, un2), (h1, h3, un1, un3))):
        hco = cid * 4 + q * 2
        ch0 = _vtake(cv16, zi + hco)
        ch1 = _vtake(cv16, zi + (hco + 1))

        def load_idx(jj, bu):
            base = ebase + jj * C1
            pltpu.sync_copy(src.at[pl.ds(base, C1)], bu[0])
            pltpu.sync_copy(dst.at[pl.ds(base, C1)], bu[1])
            pltpu.sync_copy(ea.at[pl.ds(base, C1)], bu[2])

        def fire_gathers(hc, s0t, s1t, d0t, d1t, bu):
            sem = bu[8]
            return [pltpu.async_copy(hc.at[bu[0]], bu[7], sem),
                    pltpu.async_copy(s0t.at[bu[0]], bu[3], sem),
                    pltpu.async_copy(s1t.at[bu[0]], bu[4], sem),
                    pltpu.async_copy(d0t.at[bu[1]], bu[5], sem),
                    pltpu.async_copy(d1t.at[bu[1]], bu[6], sem)]

        def compute(bu, wh):
            eav_b, asr0_b, asr1_b = bu[2], bu[3], bu[4]
            ads0_b, ads1_b, trows_b = bu[5], bu[6], bu[7]

            def grp(g, carry2):
                sl16 = pl.ds(g * 16, 16)
                ev = eav_b[sl16]
                a0 = asr0_b[sl16] + ads0_b[sl16] + ev * ch0
                a1 = asr1_b[sl16] + ads1_b[sl16] + ev * ch1
                a0 = jnp.where(a0 >= 0.0, a0, 0.2 * a0)
                a1 = jnp.where(a1 >= 0.0, a1, 0.2 * a1)
                w0 = jnp.exp(a0)
                w1 = jnp.exp(a1)
                for i in range(16):
                    e = g * 16 + i
                    si = _splat16(i)
                    wb0 = _vtake(w0, si)
                    wb1 = _vtake(w1, si)
                    wh[e, pl.ds(64, 16)] = jnp.where(
                        lane0, wb0, jnp.where(lane1, wb1, 0.0))
                    for r in range(4):
                        sl = pl.ds(r * 16, 16)
                        wh[e, sl] = trows_b[e, sl] * (wb0 if r < 2 else wb1)
                return carry2

            lax.fori_loop(0, C1 // 16, grp, 0)

        for jj in range(10):
            pltpu.sync_copy(zb, un_acc.at[pl.ds(r0 + jj * 64, 64)])
        plsc.subcore_barrier()

        variants = (
            (0, hc0, sas[2 * q], sas[2 * q + 1],
             das[2 * q], das[2 * q + 1]),
            (1, hc1, sas[4 + 2 * q], sas[4 + 2 * q + 1],
             das[4 + 2 * q], das[4 + 2 * q + 1]),
        )

        def pair(jp, carry):
            for cv, hc, s0t, s1t, d0t, d1t in variants:

                @pl.when(cid == cv)
                def _():
                    j0 = jp * 2
                    load_idx(j0, bufs[0])
                    load_idx(j0 + 1, bufs[1])
                    ha = fire_gathers(hc, s0t, s1t, d0t, d1t, bufs[0])
                    hb = fire_gathers(hc, s0t, s1t, d0t, d1t, bufs[1])
                    for h in ha:
                        h.wait()
                    compute(bufs[0], wh0)
                    s0 = pltpu.async_copy(wh0, un_acc.at[bufs[0][1]], sem_s,
                                          add=True)
                    for h in hb:
                        h.wait()
                    compute(bufs[1], wh1)
                    s1 = pltpu.async_copy(wh1, un_acc.at[bufs[1][1]], sem_s,
                                          add=True)
                    s0.wait()
                    s1.wait()
            return carry

        lax.fori_loop(0, NCH // 2, pair, 0)

        if NCH % 2 == 1:
            for cv, hc, s0t, s1t, d0t, d1t in variants:

                @pl.when(cid == cv)
                def _():
                    load_idx(NCH - 1, bufs[0])
                    ha = fire_gathers(hc, s0t, s1t, d0t, d1t, bufs[0])
                    for h in ha:
                        h.wait()
                    compute(bufs[0], wh0)
                    pltpu.sync_copy(wh0, un_acc.at[bufs[0][1]], add=True)
        plsc.subcore_barrier()

        for jj in range(5):
            rr = r0 + jj * 128

            @pl.when(cid == 0)
            def _():
                pltpu.sync_copy(un_acc.at[pl.ds(rr, 128)],
                                uo0.at[pl.ds(rr, 128)])

            @pl.when(cid == 1)
            def _():
                pltpu.sync_copy(un_acc.at[pl.ds(rr, 128)],
                                uo1.at[pl.ds(rr, 128)])


def _sc1(src, dst, ea, hp, sa_cols, da_cols, c16):
    f32 = jnp.float32
    k = pl.kernel(
        _sc1_body,
        mesh=_MESH,
        compiler_params=pltpu.CompilerParams(use_tc_tiling_on_sc=False),
        out_type=[jax.ShapeDtypeStruct((NP, 80), f32)] * 4,
        scratch_types=[
            pltpu.VMEM((64, 80), f32),
            pltpu.VMEM((16,), f32),
        ] + [
            pltpu.VMEM((C1,), jnp.int32),
            pltpu.VMEM((C1,), jnp.int32),
            pltpu.VMEM((C1,), f32),
            pltpu.VMEM((C1,), f32),
            pltpu.VMEM((C1,), f32),
            pltpu.VMEM((C1,), f32),
            pltpu.VMEM((C1,), f32),
            pltpu.VMEM((C1, 64), f32),
        ] * 2 + [
            pltpu.VMEM((C1, 80), f32),
            pltpu.VMEM((C1, 80), f32),
            pltpu.VMEM_SHARED((NP, 80), f32),
            pltpu.SemaphoreType.DMA,
            pltpu.SemaphoreType.DMA,
            pltpu.SemaphoreType.DMA,
        ],
    )
    return k(src, dst, ea, *hp, *sa_cols, *da_cols, c16)


def _sc2_body(src, dst, ea, t2, sa2, da2, c2v,
              un2_a, un2_b,
              zb, cbuf,
              srcv_0, dstv_0, eav_0, asrv_0, adsv_0, t2rows_0,
              srcv_1, dstv_1, eav_1, asrv_1, adsv_1, t2rows_1,
              wh20, wh21, un_acc, sem0, sem1, sem_s):
    f32 = jnp.float32
    z16 = jnp.zeros((16,), f32)
    cid = lax.axis_index("c")
    sid = lax.axis_index("s")
    wid = cid * 16 + sid
    iota16 = lax.iota(jnp.int32, 16)
    lane0 = iota16 == 0
    zi = jnp.zeros((16,), jnp.int32)

    pltpu.sync_copy(c2v, cbuf)
    ch = _vtake(cbuf[pl.ds(0, 16)], zi)

    def zrow(i, carry):
        for j in range(3):
            zb[i, pl.ds(j * 16, 16)] = z16
        return carry

    lax.fori_loop(0, 64, zrow, 0)

    r0 = sid * ROWS_PT
    for jj in range(10):
        pltpu.sync_copy(zb, un_acc.at[pl.ds(r0 + jj * 64, 64)])
    plsc.subcore_barrier()

    ebase = wid * EPW2
    bufs = ((srcv_0, dstv_0, eav_0, asrv_0, adsv_0, t2rows_0, sem0),
            (srcv_1, dstv_1, eav_1, asrv_1, adsv_1, t2rows_1, sem1))
    NCH = EPW2 // C2

    def load_idx(jj, bu):
        base = ebase + jj * C2
        pltpu.sync_copy(src.at[pl.ds(base, C2)], bu[0])
        pltpu.sync_copy(dst.at[pl.ds(base, C2)], bu[1])
        pltpu.sync_copy(ea.at[pl.ds(base, C2)], bu[2])

    def fire_gathers(bu):
        sem = bu[6]
        return [pltpu.async_copy(t2.at[bu[0]], bu[5], sem),
                pltpu.async_copy(sa2.at[bu[0]], bu[3], sem),
                pltpu.async_copy(da2.at[bu[1]], bu[4], sem)]

    def compute(bu, wh2):
        eav_b, asrv_b, adsv_b, t2rows_b = bu[2], bu[3], bu[4], bu[5]

        def grp(g, carry2):
            sl16 = pl.ds(g * 16, 16)
            a = asrv_b[sl16] + adsv_b[sl16] + eav_b[sl16] * ch
            a = jnp.where(a >= 0.0, a, 0.2 * a)
            w = jnp.exp(a)
            for i in range(16):
                e = g * 16 + i
                wb = _vtake(w, _splat16(i))
                wh2[e, pl.ds(32, 16)] = jnp.where(lane0, wb, 0.0)
                wh2[e, pl.ds(0, 16)] = t2rows_b[e, pl.ds(0, 16)] * wb
                wh2[e, pl.ds(16, 16)] = t2rows_b[e, pl.ds(16, 16)] * wb
            return carry2

        lax.fori_loop(0, C2 // 16, grp, 0)

    def pair(jp, carry):
        j0 = jp * 2
        load_idx(j0, bufs[0])
        load_idx(j0 + 1, bufs[1])
        ha = fire_gathers(bufs[0])
        hb = fire_gathers(bufs[1])
        for h in ha:
            h.wait()
        compute(bufs[0], wh20)
        s0 = pltpu.async_copy(wh20, un_acc.at[bufs[0][1]], sem_s, add=True)
        for h in hb:
            h.wait()
        compute(bufs[1], wh21)
        s1 = pltpu.async_copy(wh21, un_acc.at[bufs[1][1]], sem_s, add=True)
        s0.wait()
        s1.wait()
        return carry

    lax.fori_loop(0, NCH // 2, pair, 0)
    load_idx(NCH - 1, bufs[0])
    for h in fire_gathers(bufs[0]):
        h.wait()
    compute(bufs[0], wh20)
    pltpu.sync_copy(wh20, un_acc.at[bufs[0][1]], add=True)
    plsc.subcore_barrier()

    for jj in range(5):
        rr = r0 + jj * 128

        @pl.when(cid == 0)
        def _():
            pltpu.sync_copy(un_acc.at[pl.ds(rr, 128)],
                            un2_a.at[pl.ds(rr, 128)])

        @pl.when(cid == 1)
        def _():
            pltpu.sync_copy(un_acc.at[pl.ds(rr, 128)],
                            un2_b.at[pl.ds(rr, 128)])


def _sc2(src, dst, ea, t2, sa2, da2, c2v):
    f32 = jnp.float32
    k = pl.kernel(
        _sc2_body,
        mesh=_MESH,
        compiler_params=pltpu.CompilerParams(use_tc_tiling_on_sc=False),
        out_type=[jax.ShapeDtypeStruct((NP, 48), f32)] * 2,
        scratch_types=[
            pltpu.VMEM((64, 48), f32),
            pltpu.VMEM((16,), f32),
        ] + [
            pltpu.VMEM((C2,), jnp.int32),
            pltpu.VMEM((C2,), jnp.int32),
            pltpu.VMEM((C2,), f32),
            pltpu.VMEM((C2,), f32),
            pltpu.VMEM((C2,), f32),
            pltpu.VMEM((C2, 32), f32),
        ] * 2 + [
            pltpu.VMEM((C2, 48), f32),
            pltpu.VMEM((C2, 48), f32),
            pltpu.VMEM_SHARED((NP, 48), f32),
            pltpu.SemaphoreType.DMA,
            pltpu.SemaphoreType.DMA,
            pltpu.SemaphoreType.DMA,
        ],
    )
    return k(src, dst, ea, t2, sa2, da2, c2v)


def kernel(x, edge_index, edge_attr, batch, W1, a_src1, a_dst1, We1, a_edge1,
           b1, g1, be1, W2, a_src2, a_dst2, We2, a_edge2, b2, g2, be2, Wf, bf):
    f32 = jnp.float32
    src = edge_index[0]
    dst = edge_index[1]
    ea = edge_attr[:, 0]

    W1r = W1.reshape(D_IN, HEADS, HID)
    vsrc = jnp.einsum("dhj,hj->dh", W1r, a_src1)
    vdst = jnp.einsum("dhj,hj->dh", W1r, a_dst1)
    c1 = (We1.reshape(HEADS, HID) * a_edge1).sum(-1)
    wpieces = [W1r[:, 2 * p:2 * p + 2].reshape(D_IN, 64) for p in range(4)]
    c16 = jnp.zeros((16,), f32).at[0:8].set(c1)

    hp0, hp1, hp2, hp3, sa, da = _k1(x, wpieces, vsrc, vdst)
    sa_cols = [sa[:, h] for h in range(HEADS)]
    da_cols = [da[:, h] for h in range(HEADS)]
    uns = _sc1(src, dst, ea, (hp0, hp1, hp2, hp3), sa_cols, da_cols, c16)

    bs = [b1[64 * p:64 * p + 64].reshape(1, 64) for p in range(4)]
    s1, s2 = _k3a(uns, bs)
    npad = float(NP - N)
    mu = (s1[0] - npad * b1) / N
    var = (s2[0] - npad * b1 * b1) / N - mu * mu
    scale = g1 / jnp.sqrt(var + 1e-5)
    shift = be1 - mu * scale
    scs = [scale[64 * p:64 * p + 64].reshape(1, 64) for p in range(4)]
    shs = [shift[64 * p:64 * p + 64].reshape(1, 64) for p in range(4)]
    w2s = [W2[64 * p:64 * p + 64] for p in range(4)]

    a2 = jnp.zeros((HID, 8), f32).at[:, 0].set(a_src2[0])
    ad2 = jnp.zeros((HID, 8), f32).at[:, 0].set(a_dst2[0])
    c2 = (We2[0] * a_edge2[0]).sum()
    c2v = jnp.zeros((16,), f32).at[0].set(c2)

    t2, sa2, da2 = _k3b(uns, bs, scs, shs, w2s, a2, ad2)
    un2_a, un2_b = _sc2(src, dst, ea, t2, sa2[:, 0], da2[:, 0], c2v)

    b2r = b2.reshape(1, HID)
    batch_p = jnp.concatenate(
        [batch, jnp.full((NP - N,), G, jnp.int32)]).reshape(NP, 1)
    t1, t2s = _k5a(un2_a, un2_b, b2r)
    mu2 = (t1[0] - npad * b2) / N
    var2 = (t2s[0] - npad * b2 * b2) / N - mu2 * mu2
    scale2 = (g2 / jnp.sqrt(var2 + 1e-5)).reshape(1, HID)
    shift2 = (be2 - mu2 * (g2 / jnp.sqrt(var2 + 1e-5))).reshape(1, HID)

    _, _, out = _k5b(un2_a, un2_b, b2r, scale2, shift2,
                     batch_p, Wf, bf.reshape(1, 32))
    return out
